# Initial kernel scaffold; baseline (speedup 1.0000x reference)
import jax
import jax.numpy as jnp
from jax.experimental import pallas as pl


def kernel(boxes, classification):
    raise NotImplementedError("write your pallas kernel here")


# speedup vs baseline: 1.6605x; 1.6605x over previous
import jax
import jax.numpy as jnp
from jax.experimental import pallas as pl
from jax.experimental.pallas import tpu as pltpu

N = 20000
C = 20
MAXD = 300
SCORE_THR = 0.05
NMS_THR = 0.5

_INTERPRET = False


def _nms_body(boxes_ref, scores_ref, ob_ref, os_ref, ol_ref, msc_ref):
    x1 = boxes_ref[0, 0:1, :]
    y1 = boxes_ref[0, 1:2, :]
    x2 = boxes_ref[0, 2:3, :]
    y2 = boxes_ref[0, 3:4, :]
    s = scores_ref[0, :, :]
    areas = jnp.maximum(x2 - x1, 0.0) * jnp.maximum(y2 - y1, 0.0)

    iota_n = jax.lax.broadcasted_iota(jnp.int32, (1, N), 1)
    slot = jax.lax.broadcasted_iota(jnp.int32, (C, MAXD), 1)
    neg_inf = jnp.float32(-jnp.inf)

    msc_ref[:, :] = jnp.where(s > SCORE_THR, s, neg_inf)

    def body(i, carry):
        kidx, ksc = carry
        masked = msc_ref[:, :]
        m = jnp.max(masked, axis=1, keepdims=True)
        isb = masked == m
        bidx = jnp.min(jnp.where(isb, iota_n, N), axis=1, keepdims=True)
        has = m != neg_inf
        oneh = iota_n == bidx

        def pick(v):
            return jnp.sum(jnp.where(oneh, v, 0.0), axis=1, keepdims=True)

        bx1 = pick(x1); by1 = pick(y1); bx2 = pick(x2); by2 = pick(y2)
        barea = pick(areas)
        xx1 = jnp.maximum(bx1, x1); yy1 = jnp.maximum(by1, y1)
        xx2 = jnp.minimum(bx2, x2); yy2 = jnp.minimum(by2, y2)
        w = jnp.maximum(xx2 - xx1, 0.0); h = jnp.maximum(yy2 - yy1, 0.0)
        inter = w * h
        iou = inter / (barea + areas - inter + 1e-9)
        sup = (iou > NMS_THR) | oneh
        msc_ref[:, :] = jnp.where(sup & has, neg_inf, masked)
        at_i = slot == i
        kidx = jnp.where(at_i, jnp.where(has, bidx, -1), kidx)
        ksc = jnp.where(at_i, m, ksc)
        return kidx, ksc

    kidx0 = jnp.zeros((C, MAXD), jnp.int32)
    ksc0 = jnp.full((C, MAXD), neg_inf, jnp.float32)
    kidx, ksc = jax.lax.fori_loop(0, MAXD, body, (kidx0, ksc0))

    key0 = jnp.where(kidx >= 0, ksc, neg_inf)
    pos = (jax.lax.broadcasted_iota(jnp.int32, (C, MAXD), 0) * MAXD
           + jax.lax.broadcasted_iota(jnp.int32, (C, MAXD), 1))
    out_slot = jax.lax.broadcasted_iota(jnp.int32, (1, MAXD), 1)
    big = jnp.int32(C * MAXD)

    def mbody(i, carry):
        key, anch, osc, olab = carry
        g = jnp.max(key)
        p = jnp.min(jnp.where(key == g, pos, big))
        oneh = pos == p
        anchor = jnp.sum(jnp.where(oneh, kidx, 0))
        label = p // MAXD
        va = g != neg_inf
        at_i = out_slot == i
        anch = jnp.where(at_i, jnp.where(va, anchor, -1), anch)
        osc = jnp.where(at_i, jnp.where(va, g, -1.0), osc)
        olab = jnp.where(at_i, jnp.where(va, label, -1), olab)
        key = jnp.where(oneh, neg_inf, key)
        return key, anch, osc, olab

    anch0 = jnp.full((1, MAXD), -1, jnp.int32)
    osc0 = jnp.full((1, MAXD), -1.0, jnp.float32)
    olab0 = jnp.full((1, MAXD), -1, jnp.int32)
    _, anch, osc, olab = jax.lax.fori_loop(0, MAXD, mbody,
                                           (key0, anch0, osc0, olab0))

    os_ref[0, :, :] = osc
    ol_ref[0, :, :] = olab

    a_col = jnp.reshape(anch, (MAXD, 1))
    oneh_a = a_col == iota_n
    va = a_col >= 0

    def gather_coord(v):
        g = jnp.sum(jnp.where(oneh_a, v, 0.0), axis=1, keepdims=True)
        return jnp.where(va, g, -1.0)

    ob_ref[0, :, 0:1] = gather_coord(x1)
    ob_ref[0, :, 1:2] = gather_coord(y1)
    ob_ref[0, :, 2:3] = gather_coord(x2)
    ob_ref[0, :, 3:4] = gather_coord(y2)


def kernel(boxes, classification):
    B = boxes.shape[0]
    boxes_t = boxes.transpose(0, 2, 1)
    scores_t = classification.transpose(0, 2, 1)
    grid = (B,)
    ob, osc, ol = pl.pallas_call(
        _nms_body,
        grid=grid,
        in_specs=[
            pl.BlockSpec((1, 4, N), lambda b: (b, 0, 0)),
            pl.BlockSpec((1, C, N), lambda b: (b, 0, 0)),
        ],
        out_specs=[
            pl.BlockSpec((1, MAXD, 4), lambda b: (b, 0, 0)),
            pl.BlockSpec((1, 1, MAXD), lambda b: (b, 0, 0)),
            pl.BlockSpec((1, 1, MAXD), lambda b: (b, 0, 0)),
        ],
        out_shape=[
            jax.ShapeDtypeStruct((B, MAXD, 4), jnp.float32),
            jax.ShapeDtypeStruct((B, 1, MAXD), jnp.float32),
            jax.ShapeDtypeStruct((B, 1, MAXD), jnp.int32),
        ],
        scratch_shapes=[pltpu.VMEM((C, N), jnp.float32)],
        interpret=_INTERPRET,
    )(boxes_t, scores_t)
    return ob, osc.reshape(B, MAXD), ol.reshape(B, MAXD)


# speedup vs baseline: 1.9505x; 1.1746x over previous
import jax
import jax.numpy as jnp
from jax.experimental import pallas as pl
from jax.experimental.pallas import tpu as pltpu

N = 20000
C = 20
MAXD = 300
K = 512
BLK = 2000
SCORE_THR = 0.05
NMS_THR = 0.5
TH0 = 0x3D4CCCCD
HI0 = 0x3F800000

_INTERPRET = False


def _cumsum_lanes(x):
    n = x.shape[-1]
    k = 1
    while k < n:
        shifted = jnp.concatenate(
            [jnp.zeros(x.shape[:-1] + (k,), x.dtype), x[..., : n - k]], axis=-1)
        x = x + shifted
        k *= 2
    return x


def _ac_body(bits_ref, scores_ref, boxes_ref, hi_ref, wm_ref,
             cidx_ref, csc_ref, cx1_ref, cy1_ref, cx2_ref, cy2_ref,
             thi_ref, twm_ref, rem_ref,
             pos_s, chunk_s):
    bits = bits_ref[0, :, :]
    hi = hi_ref[0, :, :]
    wm = wm_ref[0, :, :]
    iota_n = jax.lax.broadcasted_iota(jnp.int32, (C, N), 1)
    cand = bits > TH0
    restricted = cand & ((bits < hi) | ((bits == hi) & (iota_n >= wm)))
    ri = restricted.astype(jnp.int32)
    cntr = jnp.sum(ri, axis=1, keepdims=True)

    def bs_body(_, lohi):
        lo, hicur = lohi
        mid = (lo + hicur) // 2
        fmid = jnp.sum((restricted & (bits > mid)).astype(jnp.int32),
                       axis=1, keepdims=True)
        take = fmid < K
        return jnp.where(take, lo, mid + 1), jnp.where(take, mid, hicur)

    lo0 = jnp.full((C, 1), TH0, jnp.int32)
    hi0 = jnp.full((C, 1), HI0, jnp.int32)
    _, tbits = jax.lax.fori_loop(0, 30, bs_body, (lo0, hi0))

    small = cntr <= K
    tbits = jnp.where(small, TH0, tbits)
    mask_hi = restricted & (bits > tbits)
    n_hi = jnp.sum(mask_hi.astype(jnp.int32), axis=1, keepdims=True)
    n_hi = jnp.where(small, cntr, n_hi)
    n_eq_take = jnp.where(small, 0, K - n_hi)

    mask_eq = restricted & (bits == tbits)
    eq_rank = _cumsum_lanes(mask_eq.astype(jnp.int32)) - mask_eq.astype(jnp.int32)
    eq_take = mask_eq & (eq_rank < n_eq_take)
    chunk = mask_hi | eq_take
    n_chunk = n_hi + n_eq_take
    next_wm = jnp.max(jnp.where(eq_take, iota_n, -1), axis=1, keepdims=True) + 1

    thi_ref[0, :, :] = tbits
    twm_ref[0, :, :] = next_wm
    rem_ref[0, :, :] = cntr - n_chunk

    ci = chunk.astype(jnp.int32)
    pos_s[:, :] = _cumsum_lanes(ci) - ci
    chunk_s[:, :] = ci

    x1 = boxes_ref[0, 0:1, :]
    y1 = boxes_ref[0, 1:2, :]
    x2 = boxes_ref[0, 2:3, :]
    y2 = boxes_ref[0, 3:4, :]
    idxf = jax.lax.broadcasted_iota(jnp.int32, (1, N), 1).astype(jnp.float32)
    iota_k = jax.lax.broadcasted_iota(jnp.int32, (K, 1), 0)
    neg_inf = jnp.float32(-jnp.inf)

    def cls_body(c, _):
        posc = pos_s[pl.ds(c, 1), :]
        chc = chunk_s[pl.ds(c, 1), :]
        sc_row = scores_ref[0, pl.ds(c, 1), :]
        acc = jnp.zeros((K, 6), jnp.float32)
        for j in range(N // BLK):
            cols = slice(j * BLK, (j + 1) * BLK)
            pb = posc[:, cols]
            cb = chc[:, cols] != 0
            oneh = ((pb == iota_k) & cb).astype(jnp.float32)
            payload = jnp.concatenate(
                [jnp.reshape(v[:, cols], (BLK, 1)) for v in
                 (idxf, sc_row, x1, y1, x2, y2)], axis=1)
            acc = acc + jnp.dot(oneh, payload,
                                precision=jax.lax.Precision.HIGHEST,
                                preferred_element_type=jnp.float32)
        accs = jnp.reshape(acc[:, 1:2], (1, K))
        cidx_ref[0, pl.ds(c, 1), :] = jnp.reshape(acc[:, 0:1], (1, K))
        csc_ref[0, pl.ds(c, 1), :] = jnp.where(accs > SCORE_THR, accs, neg_inf)
        cx1_ref[0, pl.ds(c, 1), :] = jnp.reshape(acc[:, 2:3], (1, K))
        cy1_ref[0, pl.ds(c, 1), :] = jnp.reshape(acc[:, 3:4], (1, K))
        cx2_ref[0, pl.ds(c, 1), :] = jnp.reshape(acc[:, 4:5], (1, K))
        cy2_ref[0, pl.ds(c, 1), :] = jnp.reshape(acc[:, 5:6], (1, K))
        return 0

    jax.lax.fori_loop(0, C, cls_body, 0)


def _t_body(cidx_ref, csc_ref, cx1_ref, cy1_ref, cx2_ref, cy2_ref, rem_ref,
            kidx_ref, ksc_ref, kx1_ref, ky1_ref, kx2_ref, ky2_ref, kcnt_ref,
            oidx_ref, osc_ref, ox1_ref, oy1_ref, ox2_ref, oy2_ref,
            ocnt_ref, done_ref,
            s_s, p_s):
    neg_inf = jnp.float32(-jnp.inf)
    iota300 = jax.lax.broadcasted_iota(jnp.int32, (MAXD, 1), 0).astype(jnp.float32)

    def cls_body(c, _):
        r = pl.ds(c, 1)
        sc = csc_ref[0, r, :]
        idxf = cidx_ref[0, r, :]
        x1 = cx1_ref[0, r, :]; y1 = cy1_ref[0, r, :]
        x2 = cx2_ref[0, r, :]; y2 = cy2_ref[0, r, :]
        area = jnp.maximum(x2 - x1, 0.0) * jnp.maximum(y2 - y1, 0.0)
        scT = jnp.reshape(sc, (K, 1))
        idxT = jnp.reshape(idxf, (K, 1))
        x1T = jnp.reshape(x1, (K, 1)); y1T = jnp.reshape(y1, (K, 1))
        x2T = jnp.reshape(x2, (K, 1)); y2T = jnp.reshape(y2, (K, 1))
        areaT = jnp.reshape(area, (K, 1))

        kx1 = kx1_ref[0, r, :]; ky1 = ky1_ref[0, r, :]
        kx2 = kx2_ref[0, r, :]; ky2 = ky2_ref[0, r, :]
        karea = jnp.maximum(kx2 - kx1, 0.0) * jnp.maximum(ky2 - ky1, 0.0)
        xx1 = jnp.maximum(kx1, x1T); yy1 = jnp.maximum(ky1, y1T)
        xx2 = jnp.minimum(kx2, x2T); yy2 = jnp.minimum(ky2, y2T)
        w = jnp.maximum(xx2 - xx1, 0.0); h = jnp.maximum(yy2 - yy1, 0.0)
        inter = w * h
        iouk = inter / (karea + areaT - inter + 1e-9)
        supk = jnp.sum((iouk > NMS_THR).astype(jnp.float32), axis=1,
                       keepdims=True)
        alive0 = jnp.reshape(
            jnp.where((scT != neg_inf) & (supk == 0.0), 1.0, 0.0), (1, K))

        prec = (scT > sc) | ((scT == sc) & (idxT < idxf))
        xx1c = jnp.maximum(x1T, x1); yy1c = jnp.maximum(y1T, y1)
        xx2c = jnp.minimum(x2T, x2); yy2c = jnp.minimum(y2T, y2)
        wc = jnp.maximum(xx2c - xx1c, 0.0); hc = jnp.maximum(yy2c - yy1c, 0.0)
        interc = wc * hc
        iouc = interc / (areaT + area - interc + 1e-9)
        s_s[:, :] = jnp.where(prec & (iouc > NMS_THR), 1.0, 0.0)
        p_s[:, :] = jnp.where(prec, 1.0, 0.0)

        def fp_cond(carry):
            _, changed = carry
            return changed

        def fp_body(carry):
            kv, _ = carry
            sup = jnp.dot(kv, s_s[:, :], precision=jax.lax.Precision.HIGHEST,
                          preferred_element_type=jnp.float32)
            knew = alive0 * jnp.where(sup == 0.0, 1.0, 0.0)
            return knew, jnp.any(knew != kv)

        kfin, _ = jax.lax.while_loop(fp_cond, fp_body, (alive0, True))

        rank = jnp.dot(kfin, p_s[:, :], precision=jax.lax.Precision.HIGHEST,
                       preferred_element_type=jnp.float32)
        kc = kcnt_ref[0, r, :].astype(jnp.float32)
        f = kfin * jnp.where(rank + kc < MAXD, 1.0, 0.0)
        n_new = jnp.sum(f)
        tslot = rank + kc
        oneh = jnp.where((tslot == iota300) & (f != 0.0), 1.0, 0.0)
        payload = jnp.concatenate([idxT, scT, x1T, y1T, x2T, y2T], axis=1)
        newv = jnp.dot(oneh, payload, precision=jax.lax.Precision.HIGHEST,
                       preferred_element_type=jnp.float32)
        wmask = jnp.sum(oneh, axis=1, keepdims=True) > 0.0

        def upd(out_ref, in_ref, col):
            old = in_ref[0, r, :]
            new = jnp.reshape(newv[:, col:col + 1], (1, MAXD))
            wrow = jnp.reshape(wmask, (1, MAXD))
            out_ref[0, r, :] = jnp.where(wrow, new, old)

        upd(oidx_ref, kidx_ref, 0)
        upd(osc_ref, ksc_ref, 1)
        upd(ox1_ref, kx1_ref, 2)
        upd(oy1_ref, ky1_ref, 3)
        upd(ox2_ref, kx2_ref, 4)
        upd(oy2_ref, ky2_ref, 5)
        kcn = kcnt_ref[0, r, :] + n_new.astype(jnp.int32)
        ocnt_ref[0, r, :] = kcn
        rem = rem_ref[0, r, :]
        done_ref[0, r, :] = ((kcn >= MAXD) | (rem == 0)).astype(jnp.int32)
        return 0

    jax.lax.fori_loop(0, C, cls_body, 0)


def _m_body(kidx_ref, ksc_ref, kx1_ref, ky1_ref, kx2_ref, ky2_ref,
            ob_ref, os_ref, ol_ref):
    neg_inf = jnp.float32(-jnp.inf)
    key0 = ksc_ref[0, :, :]
    kidx = kidx_ref[0, :, :].astype(jnp.int32)
    tiepos = (jax.lax.broadcasted_iota(jnp.int32, (C, MAXD), 0) * N + kidx)
    out_slot = jax.lax.broadcasted_iota(jnp.int32, (1, MAXD), 1)
    big = jnp.int32(C * N)

    kx1 = kx1_ref[0, :, :]; ky1 = ky1_ref[0, :, :]
    kx2 = kx2_ref[0, :, :]; ky2 = ky2_ref[0, :, :]

    def mbody(i, carry):
        key, osc, olab, oanch, bx1, by1, bx2, by2 = carry
        g = jnp.max(key)
        p = jnp.min(jnp.where(key == g, tiepos, big))
        oneh = (key == g) & (tiepos == p)
        va = g != neg_inf
        at_i = out_slot == i

        def ext(v):
            return jnp.sum(jnp.where(oneh, v, 0.0))

        anchor = ext(kidx.astype(jnp.float32))
        label = p // N
        osc = jnp.where(at_i, jnp.where(va, g, -1.0), osc)
        olab = jnp.where(at_i, jnp.where(va, label, -1), olab)
        oanch = jnp.where(at_i, jnp.where(va, anchor, -1.0), oanch)
        bx1 = jnp.where(at_i, jnp.where(va, ext(kx1), -1.0), bx1)
        by1 = jnp.where(at_i, jnp.where(va, ext(ky1), -1.0), by1)
        bx2 = jnp.where(at_i, jnp.where(va, ext(kx2), -1.0), bx2)
        by2 = jnp.where(at_i, jnp.where(va, ext(ky2), -1.0), by2)
        key = jnp.where(oneh, neg_inf, key)
        return key, osc, olab, oanch, bx1, by1, bx2, by2

    z = jnp.full((1, MAXD), -1.0, jnp.float32)
    zi = jnp.full((1, MAXD), -1, jnp.int32)
    carry0 = (key0, z, zi, z, z, z, z, z)
    _, osc, olab, _, bx1, by1, bx2, by2 = jax.lax.fori_loop(
        0, MAXD, mbody, carry0)

    os_ref[0, :, :] = osc
    ol_ref[0, :, :] = olab
    ob_ref[0, :, 0:1] = jnp.reshape(bx1, (MAXD, 1))
    ob_ref[0, :, 1:2] = jnp.reshape(by1, (MAXD, 1))
    ob_ref[0, :, 2:3] = jnp.reshape(bx2, (MAXD, 1))
    ob_ref[0, :, 3:4] = jnp.reshape(by2, (MAXD, 1))


def kernel(boxes, classification):
    B = boxes.shape[0]
    boxes_t = boxes.transpose(0, 2, 1)
    scores_t = classification.transpose(0, 2, 1)
    bits_t = jax.lax.bitcast_convert_type(scores_t, jnp.int32)

    def bspec(shape):
        return pl.BlockSpec((1,) + shape,
                            lambda b: (b,) + (0,) * len(shape))

    f32 = jnp.float32
    i32 = jnp.int32

    ac = pl.pallas_call(
        _ac_body,
        grid=(B,),
        in_specs=[bspec((C, N)), bspec((C, N)), bspec((4, N)),
                  bspec((C, 1)), bspec((C, 1))],
        out_specs=[bspec((C, K))] * 6 + [bspec((C, 1))] * 3,
        out_shape=[jax.ShapeDtypeStruct((B, C, K), f32)] * 6
        + [jax.ShapeDtypeStruct((B, C, 1), i32)] * 3,
        scratch_shapes=[pltpu.VMEM((C, N), i32), pltpu.VMEM((C, N), i32)],
        interpret=_INTERPRET,
    )

    tk = pl.pallas_call(
        _t_body,
        grid=(B,),
        in_specs=[bspec((C, K))] * 6 + [bspec((C, 1))]
        + [bspec((C, MAXD))] * 6 + [bspec((C, 1))],
        out_specs=[bspec((C, MAXD))] * 6 + [bspec((C, 1))] * 2,
        out_shape=[jax.ShapeDtypeStruct((B, C, MAXD), f32)] * 6
        + [jax.ShapeDtypeStruct((B, C, 1), i32)] * 2,
        scratch_shapes=[pltpu.VMEM((K, K), f32), pltpu.VMEM((K, K), f32)],
        interpret=_INTERPRET,
    )

    mg = pl.pallas_call(
        _m_body,
        grid=(B,),
        in_specs=[bspec((C, MAXD))] * 6,
        out_specs=[bspec((MAXD, 4)), bspec((1, MAXD)), bspec((1, MAXD))],
        out_shape=[jax.ShapeDtypeStruct((B, MAXD, 4), f32),
                   jax.ShapeDtypeStruct((B, 1, MAXD), f32),
                   jax.ShapeDtypeStruct((B, 1, MAXD), i32)],
        interpret=_INTERPRET,
    )

    neg_inf = jnp.float32(-jnp.inf)
    hi = jnp.full((B, C, 1), HI0, i32)
    wm = jnp.zeros((B, C, 1), i32)
    done = jnp.zeros((B, C, 1), i32)
    kplane = jnp.zeros((B, C, MAXD), f32)
    kept0 = (kplane, jnp.full((B, C, MAXD), neg_inf, f32),
             kplane, kplane, kplane, kplane)
    kcnt = jnp.zeros((B, C, 1), i32)

    def cond(st):
        return jnp.any(st[2] == 0)

    def body(st):
        hi, wm, done, kept, kcnt = st
        (cidx, csc, cx1, cy1, cx2, cy2,
         thi, twm, rem) = ac(bits_t, scores_t, boxes_t, hi, wm)
        outs = tk(cidx, csc, cx1, cy1, cx2, cy2, rem,
                  kept[0], kept[1], kept[2], kept[3], kept[4], kept[5], kcnt)
        nkept = tuple(outs[0:6])
        nkcnt, ndone = outs[6], outs[7]
        return (thi, twm, ndone, nkept, nkcnt)

    hi, wm, done, kept, kcnt = jax.lax.while_loop(
        cond, body, (hi, wm, done, kept0, kcnt))

    ob, osc, ol = mg(kept[0], kept[1], kept[2], kept[3], kept[4], kept[5])
    return ob, osc.reshape(B, MAXD), ol.reshape(B, MAXD)


# speedup vs baseline: 2.1306x; 1.0923x over previous
import jax
import jax.numpy as jnp
from jax.experimental import pallas as pl
from jax.experimental.pallas import tpu as pltpu

N = 20000
C = 20
MAXD = 300
K = 512
MK = 384
BLK = 2000
SCORE_THR = 0.05
NMS_THR = 0.5
TH0 = 0x3D4CCCCD
HI0 = 0x3F800000

_INTERPRET = False


def _cumsum_lanes(x):
    n = x.shape[-1]
    k = 1
    while k < n:
        shifted = jnp.concatenate(
            [jnp.zeros(x.shape[:-1] + (k,), x.dtype), x[..., : n - k]], axis=-1)
        x = x + shifted
        k *= 2
    return x


def _ac_body(bits_ref, scores_ref, boxes_ref, hi_ref, wm_ref,
             cidx_ref, csc_ref, cx1_ref, cy1_ref, cx2_ref, cy2_ref,
             thi_ref, twm_ref, rem_ref,
             pos_s, chunk_s):
    bits = bits_ref[0, :, :]
    hi = hi_ref[0, :, :]
    wm = wm_ref[0, :, :]
    iota_n = jax.lax.broadcasted_iota(jnp.int32, (C, N), 1)
    cand = bits > TH0
    restricted = cand & ((bits < hi) | ((bits == hi) & (iota_n >= wm)))
    ri = restricted.astype(jnp.int32)
    cntr = jnp.sum(ri, axis=1, keepdims=True)

    def bs_body(_, lohi):
        lo, hicur = lohi
        mid = (lo + hicur) // 2
        fmid = jnp.sum((restricted & (bits > mid)).astype(jnp.int32),
                       axis=1, keepdims=True)
        take = fmid < K
        return jnp.where(take, lo, mid + 1), jnp.where(take, mid, hicur)

    lo0 = jnp.full((C, 1), TH0, jnp.int32)
    hi0 = jnp.full((C, 1), HI0, jnp.int32)
    _, tbits = jax.lax.fori_loop(0, 30, bs_body, (lo0, hi0))

    small = cntr <= K
    tbits = jnp.where(small, TH0, tbits)
    mask_hi = restricted & (bits > tbits)
    n_hi = jnp.sum(mask_hi.astype(jnp.int32), axis=1, keepdims=True)
    n_hi = jnp.where(small, cntr, n_hi)
    n_eq_take = jnp.where(small, 0, K - n_hi)

    mask_eq = restricted & (bits == tbits)
    eq_rank = _cumsum_lanes(mask_eq.astype(jnp.int32)) - mask_eq.astype(jnp.int32)
    eq_take = mask_eq & (eq_rank < n_eq_take)
    chunk = mask_hi | eq_take
    n_chunk = n_hi + n_eq_take
    next_wm = jnp.max(jnp.where(eq_take, iota_n, -1), axis=1, keepdims=True) + 1

    thi_ref[0, :, :] = tbits
    twm_ref[0, :, :] = next_wm
    rem_ref[0, :, :] = cntr - n_chunk

    ci = chunk.astype(jnp.int32)
    pos_s[:, :] = _cumsum_lanes(ci) - ci
    chunk_s[:, :] = ci

    x1 = boxes_ref[0, 0:1, :]
    y1 = boxes_ref[0, 1:2, :]
    x2 = boxes_ref[0, 2:3, :]
    y2 = boxes_ref[0, 3:4, :]
    idxf = jax.lax.broadcasted_iota(jnp.int32, (1, N), 1).astype(jnp.float32)
    iota_k = jax.lax.broadcasted_iota(jnp.int32, (K, 1), 0)
    neg_inf = jnp.float32(-jnp.inf)

    def cls_body(c, _):
        posc = pos_s[pl.ds(c, 1), :]
        chc = chunk_s[pl.ds(c, 1), :]
        sc_row = scores_ref[0, pl.ds(c, 1), :]
        acc = jnp.zeros((K, 6), jnp.float32)
        for j in range(N // BLK):
            cols = slice(j * BLK, (j + 1) * BLK)
            pb = posc[:, cols]
            cb = chc[:, cols] != 0
            oneh = ((pb == iota_k) & cb).astype(jnp.float32)
            payload = jnp.concatenate(
                [jnp.reshape(v[:, cols], (BLK, 1)) for v in
                 (idxf, sc_row, x1, y1, x2, y2)], axis=1)
            acc = acc + jnp.dot(oneh, payload,
                                precision=jax.lax.Precision.HIGHEST,
                                preferred_element_type=jnp.float32)
        accs = jnp.reshape(acc[:, 1:2], (1, K))
        cidx_ref[0, pl.ds(c, 1), :] = jnp.reshape(acc[:, 0:1], (1, K))
        csc_ref[0, pl.ds(c, 1), :] = jnp.where(accs > SCORE_THR, accs, neg_inf)
        cx1_ref[0, pl.ds(c, 1), :] = jnp.reshape(acc[:, 2:3], (1, K))
        cy1_ref[0, pl.ds(c, 1), :] = jnp.reshape(acc[:, 3:4], (1, K))
        cx2_ref[0, pl.ds(c, 1), :] = jnp.reshape(acc[:, 4:5], (1, K))
        cy2_ref[0, pl.ds(c, 1), :] = jnp.reshape(acc[:, 5:6], (1, K))
        return 0

    jax.lax.fori_loop(0, C, cls_body, 0)


def _t_body(cidx_ref, csc_ref, cx1_ref, cy1_ref, cx2_ref, cy2_ref, rem_ref,
            kidx_ref, ksc_ref, kx1_ref, ky1_ref, kx2_ref, ky2_ref, kcnt_ref,
            oidx_ref, osc_ref, ox1_ref, oy1_ref, ox2_ref, oy2_ref,
            ocnt_ref, done_ref,
            s_s, p_s):
    neg_inf = jnp.float32(-jnp.inf)
    iota300 = jax.lax.broadcasted_iota(jnp.int32, (MAXD, 1), 0).astype(jnp.float32)

    def cls_body(c, _):
        r = pl.ds(c, 1)
        sc = csc_ref[0, r, :]
        idxf = cidx_ref[0, r, :]
        x1 = cx1_ref[0, r, :]; y1 = cy1_ref[0, r, :]
        x2 = cx2_ref[0, r, :]; y2 = cy2_ref[0, r, :]
        area = jnp.maximum(x2 - x1, 0.0) * jnp.maximum(y2 - y1, 0.0)
        scT = jnp.reshape(sc, (K, 1))
        idxT = jnp.reshape(idxf, (K, 1))
        x1T = jnp.reshape(x1, (K, 1)); y1T = jnp.reshape(y1, (K, 1))
        x2T = jnp.reshape(x2, (K, 1)); y2T = jnp.reshape(y2, (K, 1))
        areaT = jnp.reshape(area, (K, 1))

        kx1 = kx1_ref[0, r, :]; ky1 = ky1_ref[0, r, :]
        kx2 = kx2_ref[0, r, :]; ky2 = ky2_ref[0, r, :]
        karea = jnp.maximum(kx2 - kx1, 0.0) * jnp.maximum(ky2 - ky1, 0.0)
        xx1 = jnp.maximum(kx1, x1T); yy1 = jnp.maximum(ky1, y1T)
        xx2 = jnp.minimum(kx2, x2T); yy2 = jnp.minimum(ky2, y2T)
        w = jnp.maximum(xx2 - xx1, 0.0); h = jnp.maximum(yy2 - yy1, 0.0)
        inter = w * h
        iouk = inter / (karea + areaT - inter + 1e-9)
        supk = jnp.sum((iouk > NMS_THR).astype(jnp.float32), axis=1,
                       keepdims=True)
        alive0 = jnp.reshape(
            jnp.where((scT != neg_inf) & (supk == 0.0), 1.0, 0.0), (1, K))

        prec = (scT > sc) | ((scT == sc) & (idxT < idxf))
        xx1c = jnp.maximum(x1T, x1); yy1c = jnp.maximum(y1T, y1)
        xx2c = jnp.minimum(x2T, x2); yy2c = jnp.minimum(y2T, y2)
        wc = jnp.maximum(xx2c - xx1c, 0.0); hc = jnp.maximum(yy2c - yy1c, 0.0)
        interc = wc * hc
        iouc = interc / (areaT + area - interc + 1e-9)
        s_s[:, :] = jnp.where(prec & (iouc > NMS_THR), 1.0, 0.0)
        p_s[:, :] = jnp.where(prec, 1.0, 0.0)

        def fp_cond(carry):
            _, changed = carry
            return changed

        def fp_body(carry):
            kv, _ = carry
            sup = jnp.dot(kv, s_s[:, :], precision=jax.lax.Precision.HIGHEST,
                          preferred_element_type=jnp.float32)
            knew = alive0 * jnp.where(sup == 0.0, 1.0, 0.0)
            return knew, jnp.any(knew != kv)

        kfin, _ = jax.lax.while_loop(fp_cond, fp_body, (alive0, True))

        rank = jnp.dot(kfin, p_s[:, :], precision=jax.lax.Precision.HIGHEST,
                       preferred_element_type=jnp.float32)
        kc = kcnt_ref[0, r, :].astype(jnp.float32)
        f = kfin * jnp.where(rank + kc < MAXD, 1.0, 0.0)
        n_new = jnp.sum(f)
        tslot = rank + kc
        oneh = jnp.where((tslot == iota300) & (f != 0.0), 1.0, 0.0)
        scT_f = jnp.where(scT == neg_inf, 0.0, scT)
        payload = jnp.concatenate([idxT, scT_f, x1T, y1T, x2T, y2T], axis=1)
        newv = jnp.dot(oneh, payload, precision=jax.lax.Precision.HIGHEST,
                       preferred_element_type=jnp.float32)
        wmask = jnp.sum(oneh, axis=1, keepdims=True) > 0.0

        def upd(out_ref, in_ref, col):
            old = in_ref[0, r, :]
            new = jnp.reshape(newv[:, col:col + 1], (1, MAXD))
            wrow = jnp.reshape(wmask, (1, MAXD))
            out_ref[0, r, :] = jnp.where(wrow, new, old)

        upd(oidx_ref, kidx_ref, 0)
        upd(osc_ref, ksc_ref, 1)
        upd(ox1_ref, kx1_ref, 2)
        upd(oy1_ref, ky1_ref, 3)
        upd(ox2_ref, kx2_ref, 4)
        upd(oy2_ref, ky2_ref, 5)
        kcn = kcnt_ref[0, r, :] + n_new.astype(jnp.int32)
        ocnt_ref[0, r, :] = kcn
        rem = rem_ref[0, r, :]
        done_ref[0, r, :] = ((kcn >= MAXD) | (rem == 0)).astype(jnp.int32)
        return 0

    jax.lax.fori_loop(0, C, cls_body, 0)


def _m_body(kidx_ref, ksc_ref, ksb_ref, kx1_ref, ky1_ref, kx2_ref, ky2_ref,
            ob_ref, os_ref, ol_ref,
            pos_s, sel_s, tp_s, acc_s):
    i32 = jnp.int32
    f32 = jnp.float32
    sbits = ksb_ref[0, :, :]
    kidxf = kidx_ref[0, :, :]
    ciota = jax.lax.broadcasted_iota(i32, (C, MAXD), 0).astype(f32)
    tp_s[:, :] = ciota * jnp.float32(N) + kidxf
    valid = sbits > TH0

    nv = jnp.sum(valid.astype(i32)).reshape(1, 1)

    def bs_body(_, lohi):
        lo, hicur = lohi
        mid = (lo + hicur) // 2
        fmid = jnp.sum((sbits > mid).astype(i32)).reshape(1, 1)
        take = fmid < MAXD
        return jnp.where(take, lo, mid + 1), jnp.where(take, mid, hicur)

    lo0 = jnp.full((1, 1), TH0, i32)
    hi0 = jnp.full((1, 1), HI0, i32)
    _, tbits = jax.lax.fori_loop(0, 30, bs_body, (lo0, hi0))
    small = nv <= MAXD
    tbits = jnp.where(small, TH0, tbits)
    mask_hi = sbits > tbits
    n_hi = jnp.sum(mask_hi.astype(i32)).reshape(1, 1)
    n_hi = jnp.where(small, nv, n_hi)
    n_eq = jnp.where(small, 0, MAXD - n_hi)
    mask_eq = valid & (sbits == tbits)

    def cmprefix(m):
        le = _cumsum_lanes(m)
        row_tot = le[:, MAXD - 1:MAXD]
        ro = row_tot
        kk = 1
        while kk < C:
            sh = jnp.concatenate(
                [jnp.zeros((kk, 1), i32), ro[: C - kk, :]], axis=0)
            ro = ro + sh
            kk *= 2
        return le - m + (ro - row_tot)

    eqrank = cmprefix(mask_eq.astype(i32))
    sel = mask_hi | (mask_eq & (eqrank < n_eq))
    seli = sel.astype(i32)
    pos_s[:, :] = cmprefix(seli)
    sel_s[:, :] = seli
    acc_s[:, :] = jnp.zeros((MK, 7), f32)

    iota_mk = jax.lax.broadcasted_iota(i32, (MK, 1), 0)
    ones_col = jnp.ones((MAXD, 1), f32)

    def cls_body(c, _):
        r = pl.ds(c, 1)
        posc = pos_s[r, :]
        selc = sel_s[r, :] != 0
        oneh = ((posc == iota_mk) & selc).astype(f32)
        scrow = ksc_ref[0, r, :]
        scrow = jnp.where(scrow == jnp.float32(-jnp.inf), 0.0, scrow)
        payload = jnp.concatenate(
            [jnp.reshape(scrow, (MAXD, 1)),
             jnp.reshape(kx1_ref[0, r, :], (MAXD, 1)),
             jnp.reshape(ky1_ref[0, r, :], (MAXD, 1)),
             jnp.reshape(kx2_ref[0, r, :], (MAXD, 1)),
             jnp.reshape(ky2_ref[0, r, :], (MAXD, 1)),
             jnp.reshape(tp_s[r, :], (MAXD, 1)),
             ones_col], axis=1)
        acc_s[:, :] = acc_s[:, :] + jnp.dot(
            oneh, payload, precision=jax.lax.Precision.HIGHEST,
            preferred_element_type=f32)
        return 0

    jax.lax.fori_loop(0, C, cls_body, 0)

    acc = acc_s[:, :]
    csT = acc[:, 0:1]
    ctpT = acc[:, 5:6]
    cvaT = acc[:, 6:7] > 0.5
    cs = jnp.reshape(csT, (1, MK))
    ctp = jnp.reshape(ctpT, (1, MK))
    cva = jnp.reshape(cvaT, (1, MK))
    prec = (csT > cs) | ((csT == cs) & (ctpT < ctp))
    pf = jnp.where(prec & cvaT, 1.0, 0.0)
    rank = jnp.sum(pf, axis=0, keepdims=True)
    iota300 = jax.lax.broadcasted_iota(i32, (MAXD, 1), 0).astype(f32)
    oneh2 = jnp.where((rank == iota300) & cva, 1.0, 0.0)
    outs = jnp.dot(oneh2, acc, precision=jax.lax.Precision.HIGHEST,
                   preferred_element_type=f32)
    wm = jnp.sum(oneh2, axis=1, keepdims=True) > 0.0

    os_ref[0, :, :] = jnp.reshape(jnp.where(wm, outs[:, 0:1], -1.0), (1, MAXD))
    lab = outs[:, 5:6].astype(i32) // N
    ol_ref[0, :, :] = jnp.reshape(jnp.where(wm, lab, -1), (1, MAXD))
    ob_ref[0, :, 0:1] = jnp.where(wm, outs[:, 1:2], -1.0)
    ob_ref[0, :, 1:2] = jnp.where(wm, outs[:, 2:3], -1.0)
    ob_ref[0, :, 2:3] = jnp.where(wm, outs[:, 3:4], -1.0)
    ob_ref[0, :, 3:4] = jnp.where(wm, outs[:, 4:5], -1.0)


def kernel(boxes, classification):
    B = boxes.shape[0]
    boxes_t = boxes.transpose(0, 2, 1)
    scores_t = classification.transpose(0, 2, 1)
    bits_t = jax.lax.bitcast_convert_type(scores_t, jnp.int32)

    def bspec(shape):
        return pl.BlockSpec((1,) + shape,
                            lambda b: (b,) + (0,) * len(shape))

    f32 = jnp.float32
    i32 = jnp.int32

    ac = pl.pallas_call(
        _ac_body,
        grid=(B,),
        in_specs=[bspec((C, N)), bspec((C, N)), bspec((4, N)),
                  bspec((C, 1)), bspec((C, 1))],
        out_specs=[bspec((C, K))] * 6 + [bspec((C, 1))] * 3,
        out_shape=[jax.ShapeDtypeStruct((B, C, K), f32)] * 6
        + [jax.ShapeDtypeStruct((B, C, 1), i32)] * 3,
        scratch_shapes=[pltpu.VMEM((C, N), i32), pltpu.VMEM((C, N), i32)],
        interpret=_INTERPRET,
    )

    tk = pl.pallas_call(
        _t_body,
        grid=(B,),
        in_specs=[bspec((C, K))] * 6 + [bspec((C, 1))]
        + [bspec((C, MAXD))] * 6 + [bspec((C, 1))],
        out_specs=[bspec((C, MAXD))] * 6 + [bspec((C, 1))] * 2,
        out_shape=[jax.ShapeDtypeStruct((B, C, MAXD), f32)] * 6
        + [jax.ShapeDtypeStruct((B, C, 1), i32)] * 2,
        scratch_shapes=[pltpu.VMEM((K, K), f32), pltpu.VMEM((K, K), f32)],
        interpret=_INTERPRET,
    )

    mg = pl.pallas_call(
        _m_body,
        grid=(B,),
        in_specs=[bspec((C, MAXD))] * 7,
        out_specs=[bspec((MAXD, 4)), bspec((1, MAXD)), bspec((1, MAXD))],
        out_shape=[jax.ShapeDtypeStruct((B, MAXD, 4), f32),
                   jax.ShapeDtypeStruct((B, 1, MAXD), f32),
                   jax.ShapeDtypeStruct((B, 1, MAXD), i32)],
        scratch_shapes=[pltpu.VMEM((C, MAXD), i32), pltpu.VMEM((C, MAXD), i32),
                        pltpu.VMEM((C, MAXD), f32), pltpu.VMEM((MK, 7), f32)],
        interpret=_INTERPRET,
    )

    neg_inf = jnp.float32(-jnp.inf)
    hi = jnp.full((B, C, 1), HI0, i32)
    wm = jnp.zeros((B, C, 1), i32)
    done = jnp.zeros((B, C, 1), i32)
    kplane = jnp.zeros((B, C, MAXD), f32)
    kept0 = (kplane, jnp.full((B, C, MAXD), neg_inf, f32),
             kplane, kplane, kplane, kplane)
    kcnt = jnp.zeros((B, C, 1), i32)

    def cond(st):
        return jnp.any(st[2] == 0)

    def body(st):
        hi, wm, done, kept, kcnt = st
        (cidx, csc, cx1, cy1, cx2, cy2,
         thi, twm, rem) = ac(bits_t, scores_t, boxes_t, hi, wm)
        outs = tk(cidx, csc, cx1, cy1, cx2, cy2, rem,
                  kept[0], kept[1], kept[2], kept[3], kept[4], kept[5], kcnt)
        nkept = tuple(outs[0:6])
        nkcnt, ndone = outs[6], outs[7]
        return (thi, twm, ndone, nkept, nkcnt)

    hi, wm, done, kept, kcnt = jax.lax.while_loop(
        cond, body, (hi, wm, done, kept0, kcnt))

    ksb = jax.lax.bitcast_convert_type(kept[1], i32)
    ob, osc, ol = mg(kept[0], kept[1], ksb, kept[2], kept[3], kept[4], kept[5])
    return ob, osc.reshape(B, MAXD), ol.reshape(B, MAXD)


# speedup vs baseline: 2.7628x; 1.2967x over previous
import jax
import jax.numpy as jnp
from jax.experimental import pallas as pl
from jax.experimental.pallas import tpu as pltpu

N = 20000
C = 20
MAXD = 300
K = 512
MK = 384
BLK = 2000
SCORE_THR = 0.05
NMS_THR = 0.5
TH0 = 0x3D4CCCCD
HI0 = 0x3F800000

_INTERPRET = False


def _cumsum_lanes(x):
    n = x.shape[-1]
    k = 1
    while k < n:
        shifted = jnp.concatenate(
            [jnp.zeros(x.shape[:-1] + (k,), x.dtype), x[..., : n - k]], axis=-1)
        x = x + shifted
        k *= 2
    return x


def _bf16_split3(v):
    h = v.astype(jnp.bfloat16).astype(jnp.float32)
    r = v - h
    m = r.astype(jnp.bfloat16).astype(jnp.float32)
    return h, m, r - m


def _split_cols(p):
    h, m, l = _bf16_split3(p)
    return jnp.concatenate([h, m, l], axis=1)


def _ac_body(bits_ref, scores_ref, boxes_ref, hi_ref, wm_ref,
             cidx_ref, csc_ref, cx1_ref, cy1_ref, cx2_ref, cy2_ref,
             thi_ref, twm_ref, rem_ref,
             pos_s, chunk_s):
    bits = bits_ref[0, :, :]
    hi = hi_ref[0, :, :]
    wm = wm_ref[0, :, :]
    iota_n = jax.lax.broadcasted_iota(jnp.int32, (C, N), 1)
    cand = bits > TH0
    restricted = cand & ((bits < hi) | ((bits == hi) & (iota_n >= wm)))
    ri = restricted.astype(jnp.int32)
    cntr = jnp.sum(ri, axis=1, keepdims=True)

    def bs_body(_, lohi):
        lo, hicur = lohi
        mid = (lo + hicur) // 2
        fmid = jnp.sum((restricted & (bits > mid)).astype(jnp.int32),
                       axis=1, keepdims=True)
        take = fmid < K
        return jnp.where(take, lo, mid + 1), jnp.where(take, mid, hicur)

    lo0 = jnp.full((C, 1), TH0, jnp.int32)
    hi0 = jnp.full((C, 1), HI0, jnp.int32)
    _, tbits = jax.lax.fori_loop(0, 30, bs_body, (lo0, hi0))

    small = cntr <= K
    tbits = jnp.where(small, TH0, tbits)
    mask_hi = restricted & (bits > tbits)
    n_hi = jnp.sum(mask_hi.astype(jnp.int32), axis=1, keepdims=True)
    n_hi = jnp.where(small, cntr, n_hi)
    n_eq_take = jnp.where(small, 0, K - n_hi)

    mask_eq = restricted & (bits == tbits)
    eq_rank = _cumsum_lanes(mask_eq.astype(jnp.int32)) - mask_eq.astype(jnp.int32)
    eq_take = mask_eq & (eq_rank < n_eq_take)
    chunk = mask_hi | eq_take
    n_chunk = n_hi + n_eq_take
    next_wm = jnp.max(jnp.where(eq_take, iota_n, -1), axis=1, keepdims=True) + 1

    thi_ref[0, :, :] = tbits
    twm_ref[0, :, :] = next_wm
    rem_ref[0, :, :] = cntr - n_chunk

    ci = chunk.astype(jnp.int32)
    pos_s[:, :] = _cumsum_lanes(ci) - ci
    chunk_s[:, :] = ci

    x1 = boxes_ref[0, 0:1, :]
    y1 = boxes_ref[0, 1:2, :]
    x2 = boxes_ref[0, 2:3, :]
    y2 = boxes_ref[0, 3:4, :]
    idxf = jax.lax.broadcasted_iota(jnp.int32, (1, N), 1).astype(jnp.float32)
    iota_k = jax.lax.broadcasted_iota(jnp.int32, (K, 1), 0)
    neg_inf = jnp.float32(-jnp.inf)

    def cls_body(c, _):
        posc = pos_s[pl.ds(c, 1), :]
        chc = chunk_s[pl.ds(c, 1), :]
        sc_row = scores_ref[0, pl.ds(c, 1), :]
        acc18 = jnp.zeros((K, 18), jnp.float32)
        for j in range(N // BLK):
            cols = slice(j * BLK, (j + 1) * BLK)
            pb = posc[:, cols]
            cb = chc[:, cols] != 0
            oneh = ((pb == iota_k) & cb).astype(jnp.float32)
            payload = jnp.concatenate(
                [jnp.reshape(v[:, cols], (BLK, 1)) for v in
                 (idxf, sc_row, x1, y1, x2, y2)], axis=1)
            acc18 = acc18 + jnp.dot(oneh, _split_cols(payload),
                                    preferred_element_type=jnp.float32)
        acc = acc18[:, 0:6] + acc18[:, 6:12] + acc18[:, 12:18]
        accs = jnp.reshape(acc[:, 1:2], (1, K))
        cidx_ref[0, pl.ds(c, 1), :] = jnp.reshape(acc[:, 0:1], (1, K))
        csc_ref[0, pl.ds(c, 1), :] = jnp.where(accs > SCORE_THR, accs, neg_inf)
        cx1_ref[0, pl.ds(c, 1), :] = jnp.reshape(acc[:, 2:3], (1, K))
        cy1_ref[0, pl.ds(c, 1), :] = jnp.reshape(acc[:, 3:4], (1, K))
        cx2_ref[0, pl.ds(c, 1), :] = jnp.reshape(acc[:, 4:5], (1, K))
        cy2_ref[0, pl.ds(c, 1), :] = jnp.reshape(acc[:, 5:6], (1, K))
        return 0

    jax.lax.fori_loop(0, C, cls_body, 0)


def _t_body(cidx_ref, csc_ref, cx1_ref, cy1_ref, cx2_ref, cy2_ref, rem_ref,
            kidx_ref, ksc_ref, kx1_ref, ky1_ref, kx2_ref, ky2_ref, kcnt_ref,
            oidx_ref, osc_ref, ox1_ref, oy1_ref, ox2_ref, oy2_ref,
            ocnt_ref, done_ref,
            s_s, p_s):
    neg_inf = jnp.float32(-jnp.inf)
    iota300 = jax.lax.broadcasted_iota(jnp.int32, (MAXD, 1), 0).astype(jnp.float32)

    def cls_body(c, _):
        r = pl.ds(c, 1)
        sc = csc_ref[0, r, :]
        idxf = cidx_ref[0, r, :]
        x1 = cx1_ref[0, r, :]; y1 = cy1_ref[0, r, :]
        x2 = cx2_ref[0, r, :]; y2 = cy2_ref[0, r, :]
        area = jnp.maximum(x2 - x1, 0.0) * jnp.maximum(y2 - y1, 0.0)
        scT = jnp.reshape(sc, (K, 1))
        idxT = jnp.reshape(idxf, (K, 1))
        x1T = jnp.reshape(x1, (K, 1)); y1T = jnp.reshape(y1, (K, 1))
        x2T = jnp.reshape(x2, (K, 1)); y2T = jnp.reshape(y2, (K, 1))
        areaT = jnp.reshape(area, (K, 1))

        kx1 = kx1_ref[0, r, :]; ky1 = ky1_ref[0, r, :]
        kx2 = kx2_ref[0, r, :]; ky2 = ky2_ref[0, r, :]
        karea = jnp.maximum(kx2 - kx1, 0.0) * jnp.maximum(ky2 - ky1, 0.0)
        xx1 = jnp.maximum(kx1, x1T); yy1 = jnp.maximum(ky1, y1T)
        xx2 = jnp.minimum(kx2, x2T); yy2 = jnp.minimum(ky2, y2T)
        w = jnp.maximum(xx2 - xx1, 0.0); h = jnp.maximum(yy2 - yy1, 0.0)
        inter = w * h
        iouk = inter / (karea + areaT - inter + 1e-9)
        supk = jnp.sum((iouk > NMS_THR).astype(jnp.float32), axis=1,
                       keepdims=True)
        alive0 = jnp.reshape(
            jnp.where((scT != neg_inf) & (supk == 0.0), 1.0, 0.0), (1, K))

        prec = (scT > sc) | ((scT == sc) & (idxT < idxf))
        xx1c = jnp.maximum(x1T, x1); yy1c = jnp.maximum(y1T, y1)
        xx2c = jnp.minimum(x2T, x2); yy2c = jnp.minimum(y2T, y2)
        wc = jnp.maximum(xx2c - xx1c, 0.0); hc = jnp.maximum(yy2c - yy1c, 0.0)
        interc = wc * hc
        iouc = interc / (areaT + area - interc + 1e-9)
        s_s[:, :] = jnp.where(prec & (iouc > NMS_THR), 1.0, 0.0)
        p_s[:, :] = jnp.where(prec, 1.0, 0.0)

        def fp_cond(carry):
            _, changed = carry
            return changed

        def fp_body(carry):
            kv, _ = carry
            sup = jnp.dot(kv, s_s[:, :], preferred_element_type=jnp.float32)
            knew = alive0 * jnp.where(sup == 0.0, 1.0, 0.0)
            return knew, jnp.any(knew != kv)

        kfin, _ = jax.lax.while_loop(fp_cond, fp_body, (alive0, True))

        rank = jnp.dot(kfin, p_s[:, :], preferred_element_type=jnp.float32)
        kc = kcnt_ref[0, r, :].astype(jnp.float32)
        f = kfin * jnp.where(rank + kc < MAXD, 1.0, 0.0)
        n_new = jnp.sum(f)
        tslot = rank + kc
        oneh = jnp.where((tslot == iota300) & (f != 0.0), 1.0, 0.0)
        scT_f = jnp.where(scT == neg_inf, 0.0, scT)
        payload = jnp.concatenate([idxT, scT_f, x1T, y1T, x2T, y2T], axis=1)
        newv18 = jnp.dot(oneh, _split_cols(payload),
                         preferred_element_type=jnp.float32)
        newv = newv18[:, 0:6] + newv18[:, 6:12] + newv18[:, 12:18]
        wmask = jnp.sum(oneh, axis=1, keepdims=True) > 0.0

        def upd(out_ref, in_ref, col):
            old = in_ref[0, r, :]
            new = jnp.reshape(newv[:, col:col + 1], (1, MAXD))
            wrow = jnp.reshape(wmask, (1, MAXD))
            out_ref[0, r, :] = jnp.where(wrow, new, old)

        upd(oidx_ref, kidx_ref, 0)
        upd(osc_ref, ksc_ref, 1)
        upd(ox1_ref, kx1_ref, 2)
        upd(oy1_ref, ky1_ref, 3)
        upd(ox2_ref, kx2_ref, 4)
        upd(oy2_ref, ky2_ref, 5)
        kcn = kcnt_ref[0, r, :] + n_new.astype(jnp.int32)
        ocnt_ref[0, r, :] = kcn
        rem = rem_ref[0, r, :]
        done_ref[0, r, :] = ((kcn >= MAXD) | (rem == 0)).astype(jnp.int32)
        return 0

    jax.lax.fori_loop(0, C, cls_body, 0)


def _m_body(kidx_ref, ksc_ref, ksb_ref, kx1_ref, ky1_ref, kx2_ref, ky2_ref,
            ob_ref, os_ref, ol_ref,
            pos_s, sel_s, tp_s, acc_s):
    i32 = jnp.int32
    f32 = jnp.float32
    sbits = ksb_ref[0, :, :]
    kidxf = kidx_ref[0, :, :]
    ciota = jax.lax.broadcasted_iota(i32, (C, MAXD), 0).astype(f32)
    tp_s[:, :] = ciota * jnp.float32(N) + kidxf
    valid = sbits > TH0

    nv = jnp.sum(valid.astype(i32)).reshape(1, 1)

    def bs_body(_, lohi):
        lo, hicur = lohi
        mid = (lo + hicur) // 2
        fmid = jnp.sum((sbits > mid).astype(i32)).reshape(1, 1)
        take = fmid < MAXD
        return jnp.where(take, lo, mid + 1), jnp.where(take, mid, hicur)

    lo0 = jnp.full((1, 1), TH0, i32)
    hi0 = jnp.full((1, 1), HI0, i32)
    _, tbits = jax.lax.fori_loop(0, 30, bs_body, (lo0, hi0))
    small = nv <= MAXD
    tbits = jnp.where(small, TH0, tbits)
    mask_hi = sbits > tbits
    n_hi = jnp.sum(mask_hi.astype(i32)).reshape(1, 1)
    n_hi = jnp.where(small, nv, n_hi)
    n_eq = jnp.where(small, 0, MAXD - n_hi)
    mask_eq = valid & (sbits == tbits)

    def cmprefix(m):
        le = _cumsum_lanes(m)
        row_tot = le[:, MAXD - 1:MAXD]
        ro = row_tot
        kk = 1
        while kk < C:
            sh = jnp.concatenate(
                [jnp.zeros((kk, 1), i32), ro[: C - kk, :]], axis=0)
            ro = ro + sh
            kk *= 2
        return le - m + (ro - row_tot)

    eqrank = cmprefix(mask_eq.astype(i32))
    sel = mask_hi | (mask_eq & (eqrank < n_eq))
    seli = sel.astype(i32)
    pos_s[:, :] = cmprefix(seli)
    sel_s[:, :] = seli
    acc_s[:, :] = jnp.zeros((MK, 21), f32)

    iota_mk = jax.lax.broadcasted_iota(i32, (MK, 1), 0)
    ones_col = jnp.ones((MAXD, 1), f32)

    def cls_body(c, _):
        r = pl.ds(c, 1)
        posc = pos_s[r, :]
        selc = sel_s[r, :] != 0
        oneh = ((posc == iota_mk) & selc).astype(f32)
        scrow = ksc_ref[0, r, :]
        scrow = jnp.where(scrow == jnp.float32(-jnp.inf), 0.0, scrow)
        payload = jnp.concatenate(
            [jnp.reshape(scrow, (MAXD, 1)),
             jnp.reshape(kx1_ref[0, r, :], (MAXD, 1)),
             jnp.reshape(ky1_ref[0, r, :], (MAXD, 1)),
             jnp.reshape(kx2_ref[0, r, :], (MAXD, 1)),
             jnp.reshape(ky2_ref[0, r, :], (MAXD, 1)),
             jnp.reshape(tp_s[r, :], (MAXD, 1)),
             ones_col], axis=1)
        acc_s[:, :] = acc_s[:, :] + jnp.dot(
            oneh, _split_cols(payload), preferred_element_type=f32)
        return 0

    jax.lax.fori_loop(0, C, cls_body, 0)

    acc21 = acc_s[:, :]
    acc = acc21[:, 0:7] + acc21[:, 7:14] + acc21[:, 14:21]
    csT = acc[:, 0:1]
    ctpT = acc[:, 5:6]
    cvaT = acc[:, 6:7] > 0.5
    cs = jnp.reshape(csT, (1, MK))
    ctp = jnp.reshape(ctpT, (1, MK))
    cva = jnp.reshape(cvaT, (1, MK))
    prec = (csT > cs) | ((csT == cs) & (ctpT < ctp))
    pf = jnp.where(prec & cvaT, 1.0, 0.0)
    rank = jnp.sum(pf, axis=0, keepdims=True)
    iota300 = jax.lax.broadcasted_iota(i32, (MAXD, 1), 0).astype(f32)
    oneh2 = jnp.where((rank == iota300) & cva, 1.0, 0.0)
    outs21 = jnp.dot(oneh2, acc21, preferred_element_type=f32)
    outs = outs21[:, 0:7] + outs21[:, 7:14] + outs21[:, 14:21]
    wm = jnp.sum(oneh2, axis=1, keepdims=True) > 0.0

    os_ref[0, :, :] = jnp.reshape(jnp.where(wm, outs[:, 0:1], -1.0), (1, MAXD))
    lab = outs[:, 5:6].astype(i32) // N
    ol_ref[0, :, :] = jnp.reshape(jnp.where(wm, lab, -1), (1, MAXD))
    ob_ref[0, :, 0:1] = jnp.where(wm, outs[:, 1:2], -1.0)
    ob_ref[0, :, 1:2] = jnp.where(wm, outs[:, 2:3], -1.0)
    ob_ref[0, :, 2:3] = jnp.where(wm, outs[:, 3:4], -1.0)
    ob_ref[0, :, 3:4] = jnp.where(wm, outs[:, 4:5], -1.0)


def kernel(boxes, classification):
    B = boxes.shape[0]
    boxes_t = boxes.transpose(0, 2, 1)
    scores_t = classification.transpose(0, 2, 1)
    bits_t = jax.lax.bitcast_convert_type(scores_t, jnp.int32)

    def bspec(shape):
        return pl.BlockSpec((1,) + shape,
                            lambda b: (b,) + (0,) * len(shape))

    f32 = jnp.float32
    i32 = jnp.int32

    ac = pl.pallas_call(
        _ac_body,
        grid=(B,),
        in_specs=[bspec((C, N)), bspec((C, N)), bspec((4, N)),
                  bspec((C, 1)), bspec((C, 1))],
        out_specs=[bspec((C, K))] * 6 + [bspec((C, 1))] * 3,
        out_shape=[jax.ShapeDtypeStruct((B, C, K), f32)] * 6
        + [jax.ShapeDtypeStruct((B, C, 1), i32)] * 3,
        scratch_shapes=[pltpu.VMEM((C, N), i32), pltpu.VMEM((C, N), i32)],
        interpret=_INTERPRET,
    )

    tk = pl.pallas_call(
        _t_body,
        grid=(B,),
        in_specs=[bspec((C, K))] * 6 + [bspec((C, 1))]
        + [bspec((C, MAXD))] * 6 + [bspec((C, 1))],
        out_specs=[bspec((C, MAXD))] * 6 + [bspec((C, 1))] * 2,
        out_shape=[jax.ShapeDtypeStruct((B, C, MAXD), f32)] * 6
        + [jax.ShapeDtypeStruct((B, C, 1), i32)] * 2,
        scratch_shapes=[pltpu.VMEM((K, K), f32), pltpu.VMEM((K, K), f32)],
        interpret=_INTERPRET,
    )

    mg = pl.pallas_call(
        _m_body,
        grid=(B,),
        in_specs=[bspec((C, MAXD))] * 7,
        out_specs=[bspec((MAXD, 4)), bspec((1, MAXD)), bspec((1, MAXD))],
        out_shape=[jax.ShapeDtypeStruct((B, MAXD, 4), f32),
                   jax.ShapeDtypeStruct((B, 1, MAXD), f32),
                   jax.ShapeDtypeStruct((B, 1, MAXD), i32)],
        scratch_shapes=[pltpu.VMEM((C, MAXD), i32), pltpu.VMEM((C, MAXD), i32),
                        pltpu.VMEM((C, MAXD), f32), pltpu.VMEM((MK, 21), f32)],
        interpret=_INTERPRET,
    )

    neg_inf = jnp.float32(-jnp.inf)
    hi = jnp.full((B, C, 1), HI0, i32)
    wm = jnp.zeros((B, C, 1), i32)
    done = jnp.zeros((B, C, 1), i32)
    kplane = jnp.zeros((B, C, MAXD), f32)
    kept0 = (kplane, jnp.full((B, C, MAXD), neg_inf, f32),
             kplane, kplane, kplane, kplane)
    kcnt = jnp.zeros((B, C, 1), i32)

    def cond(st):
        return jnp.any(st[2] == 0)

    def body(st):
        hi, wm, done, kept, kcnt = st
        (cidx, csc, cx1, cy1, cx2, cy2,
         thi, twm, rem) = ac(bits_t, scores_t, boxes_t, hi, wm)
        outs = tk(cidx, csc, cx1, cy1, cx2, cy2, rem,
                  kept[0], kept[1], kept[2], kept[3], kept[4], kept[5], kcnt)
        nkept = tuple(outs[0:6])
        nkcnt, ndone = outs[6], outs[7]
        return (thi, twm, ndone, nkept, nkcnt)

    hi, wm, done, kept, kcnt = jax.lax.while_loop(
        cond, body, (hi, wm, done, kept0, kcnt))

    ksb = jax.lax.bitcast_convert_type(kept[1], i32)
    ob, osc, ol = mg(kept[0], kept[1], ksb, kept[2], kept[3], kept[4], kept[5])
    return ob, osc.reshape(B, MAXD), ol.reshape(B, MAXD)


# speedup vs baseline: 5.3571x; 1.9390x over previous
import functools

import jax
import jax.numpy as jnp
from jax import lax
from jax.experimental import pallas as pl
from jax.experimental.pallas import tpu as pltpu
from jax.experimental.pallas import tpu_sc as plsc

N = 20000
C = 20
MAXD = 300
K = 512
MK = 384
BLK = 2000
SCORE_THR = 0.05
NMS_THR = 0.5
TH0 = 0x3D4CCCCD
HI0 = 0x3F800000

_INTERPRET = False


def _cumsum_lanes(x):
    n = x.shape[-1]
    k = 1
    while k < n:
        shifted = jnp.concatenate(
            [jnp.zeros(x.shape[:-1] + (k,), x.dtype), x[..., : n - k]], axis=-1)
        x = x + shifted
        k *= 2
    return x


def _bf16_split3(v):
    h = v.astype(jnp.bfloat16).astype(jnp.float32)
    r = v - h
    m = r.astype(jnp.bfloat16).astype(jnp.float32)
    return h, m, r - m


def _split_cols(p):
    h, m, l = _bf16_split3(p)
    return jnp.concatenate([h, m, l], axis=1)


def _ac_body(bits_ref, hi_ref, wm_ref,
             thi_ref, twm_ref, rem_ref, neq_ref):
    bits = bits_ref[0, :, :]
    hi = hi_ref[0, :, :]
    wm = wm_ref[0, :, :]
    iota_n = jax.lax.broadcasted_iota(jnp.int32, (C, N), 1)
    cand = bits > TH0
    restricted = cand & ((bits < hi) | ((bits == hi) & (iota_n >= wm)))
    ri = restricted.astype(jnp.int32)
    cntr = jnp.sum(ri, axis=1, keepdims=True)

    def bs_body(_, lohi):
        lo, hicur = lohi
        mid = (lo + hicur) // 2
        fmid = jnp.sum((restricted & (bits > mid)).astype(jnp.int32),
                       axis=1, keepdims=True)
        take = fmid < K
        return jnp.where(take, lo, mid + 1), jnp.where(take, mid, hicur)

    lo0 = jnp.full((C, 1), TH0, jnp.int32)
    hi0 = jnp.full((C, 1), HI0, jnp.int32)
    _, tbits = jax.lax.fori_loop(0, 30, bs_body, (lo0, hi0))

    small = cntr <= K
    tbits = jnp.where(small, TH0, tbits)
    mask_hi = restricted & (bits > tbits)
    n_hi = jnp.sum(mask_hi.astype(jnp.int32), axis=1, keepdims=True)
    n_hi = jnp.where(small, cntr, n_hi)
    n_eq_take = jnp.where(small, 0, K - n_hi)

    mask_eq = restricted & (bits == tbits)
    eq_rank = _cumsum_lanes(mask_eq.astype(jnp.int32)) - mask_eq.astype(jnp.int32)
    eq_take = mask_eq & (eq_rank < n_eq_take)
    chunk = mask_hi | eq_take
    n_chunk = n_hi + n_eq_take
    next_wm = jnp.max(jnp.where(eq_take, iota_n, -1), axis=1, keepdims=True) + 1

    thi_ref[0, :, :] = tbits
    twm_ref[0, :, :] = next_wm
    rem_ref[0, :, :] = cntr - n_chunk

    neq_ref[0, :, :] = n_eq_take


_NC = 2
_NS = 16
_NW = _NC * _NS
_KPAD = K + 32


@functools.lru_cache(maxsize=None)
def _make_sc_compact(BC):
    mesh = plsc.VectorSubcoreMesh(core_axis_name="c", subcore_axis_name="s",
                                  num_cores=_NC)
    f32 = jnp.float32
    i32 = jnp.int32

    @functools.partial(
        pl.kernel, mesh=mesh,
        compiler_params=pltpu.CompilerParams(needs_layout_passes=False,
                                             use_tc_tiling_on_sc=False),
        out_type=[jax.ShapeDtypeStruct((BC, K), f32),
                  jax.ShapeDtypeStruct((BC, K, 16), f32),
                  jax.ShapeDtypeStruct((BC, 16), i32)],
        scratch_types=[pltpu.VMEM((N,), f32),
                       pltpu.VMEM((_KPAD,), i32),
                       pltpu.VMEM((K,), i32),
                       pltpu.VMEM((K,), f32),
                       pltpu.VMEM((K, 16), f32),
                       pltpu.VMEM((64,), i32),
                       pltpu.SemaphoreType.DMA],
    )
    def sc_compact(scores_hbm, pay_hbm, par_hbm,
                   cidx_out, pay_out, ncnt_out,
                   sc_v, ci_v, giv, cif_v, pg_v, pv, sem):
        wid = lax.axis_index("s") * _NC + lax.axis_index("c")
        iota16 = lax.broadcasted_iota(i32, (16,), 0)
        zero16 = jnp.zeros((16,), i32)
        th016 = jnp.full((16,), TH0, i32)

        def process(t):
            pltpu.sync_copy(scores_hbm.at[t], sc_v)
            pltpu.sync_copy(par_hbm.at[t], pv)
            tb = pv[pl.ds(0, 16)]
            hi = pv[pl.ds(16, 16)]
            wm = pv[pl.ds(32, 16)]
            ne = pv[pl.ds(48, 16)]

            def initb(i, _):
                ci_v[pl.ds(i * 16, 16)] = zero16
                return 0
            lax.fori_loop(0, _KPAD // 16, initb, 0)

            def p1(i, cnt):
                v = plsc.bitcast(sc_v[pl.ds(i * 16, 16)], i32)
                gidx = iota16 + i * 16
                m = (v > tb) & ((v < hi) | ((v == hi) & (gidx >= wm)))
                plsc.store_compressed(ci_v.at[pl.ds(cnt, 16)], gidx, mask=m)
                return cnt + jnp.sum(m.astype(i32))
            cnt1 = lax.fori_loop(0, N // 16, p1, jnp.int32(0))

            def p2(i, cnt):
                v = plsc.bitcast(sc_v[pl.ds(i * 16, 16)], i32)
                gidx = iota16 + i * 16
                me = ((v > th016) & (v == tb)
                      & ((v < hi) | ((v == hi) & (gidx >= wm))))
                pre = plsc.cumsum(me.astype(i32))
                take = me & ((cnt - cnt1 + pre) <= ne)
                plsc.store_compressed(ci_v.at[pl.ds(cnt, 16)], gidx, mask=take)
                return cnt + jnp.sum(take.astype(i32))
            cnt2 = lax.fori_loop(0, N // 16, p2, cnt1)

            def g1(j, _):
                iv = ci_v[pl.ds(j * 16, 16)]
                cif_v[pl.ds(j * 16, 16)] = iv.astype(f32)
                giv[pl.ds(j * 16, 16)] = iv
                return 0
            lax.fori_loop(0, K // 16, g1, 0)

            pltpu.async_copy(pay_hbm.at[t].at[giv], pg_v, sem).wait()
            pltpu.sync_copy(cif_v, cidx_out.at[t])
            pltpu.sync_copy(pg_v, pay_out.at[t])
            ncv = jnp.full((16,), cnt2, i32)
            ci_v[pl.ds(0, 16)] = ncv
            pltpu.sync_copy(ci_v.at[pl.ds(0, 16)], ncnt_out.at[t])

        process(wid)

        @pl.when(wid + _NW < BC)
        def _second():
            process(wid + _NW)

    return sc_compact


def _t_body(cidx_ref, csc_ref, cx1_ref, cy1_ref, cx2_ref, cy2_ref, rem_ref,
            kidx_ref, ksc_ref, kx1_ref, ky1_ref, kx2_ref, ky2_ref, kcnt_ref,
            oidx_ref, osc_ref, ox1_ref, oy1_ref, ox2_ref, oy2_ref,
            ocnt_ref, done_ref,
            s_s, p_s):
    neg_inf = jnp.float32(-jnp.inf)
    iota300 = jax.lax.broadcasted_iota(jnp.int32, (MAXD, 1), 0).astype(jnp.float32)

    def cls_body(c, _):
        r = pl.ds(c, 1)
        sc = csc_ref[0, r, :]
        idxf = cidx_ref[0, r, :]
        x1 = cx1_ref[0, r, :]; y1 = cy1_ref[0, r, :]
        x2 = cx2_ref[0, r, :]; y2 = cy2_ref[0, r, :]
        area = jnp.maximum(x2 - x1, 0.0) * jnp.maximum(y2 - y1, 0.0)
        scT = jnp.reshape(sc, (K, 1))
        idxT = jnp.reshape(idxf, (K, 1))
        x1T = jnp.reshape(x1, (K, 1)); y1T = jnp.reshape(y1, (K, 1))
        x2T = jnp.reshape(x2, (K, 1)); y2T = jnp.reshape(y2, (K, 1))
        areaT = jnp.reshape(area, (K, 1))

        kx1 = kx1_ref[0, r, :]; ky1 = ky1_ref[0, r, :]
        kx2 = kx2_ref[0, r, :]; ky2 = ky2_ref[0, r, :]
        karea = jnp.maximum(kx2 - kx1, 0.0) * jnp.maximum(ky2 - ky1, 0.0)
        xx1 = jnp.maximum(kx1, x1T); yy1 = jnp.maximum(ky1, y1T)
        xx2 = jnp.minimum(kx2, x2T); yy2 = jnp.minimum(ky2, y2T)
        w = jnp.maximum(xx2 - xx1, 0.0); h = jnp.maximum(yy2 - yy1, 0.0)
        inter = w * h
        iouk = inter / (karea + areaT - inter + 1e-9)
        supk = jnp.sum((iouk > NMS_THR).astype(jnp.float32), axis=1,
                       keepdims=True)
        alive0 = jnp.reshape(
            jnp.where((scT != neg_inf) & (supk == 0.0), 1.0, 0.0), (1, K))

        prec = (scT > sc) | ((scT == sc) & (idxT < idxf))
        xx1c = jnp.maximum(x1T, x1); yy1c = jnp.maximum(y1T, y1)
        xx2c = jnp.minimum(x2T, x2); yy2c = jnp.minimum(y2T, y2)
        wc = jnp.maximum(xx2c - xx1c, 0.0); hc = jnp.maximum(yy2c - yy1c, 0.0)
        interc = wc * hc
        iouc = interc / (areaT + area - interc + 1e-9)
        s_s[:, :] = jnp.where(prec & (iouc > NMS_THR), 1.0, 0.0)
        p_s[:, :] = jnp.where(prec, 1.0, 0.0)

        def fp_cond(carry):
            _, changed = carry
            return changed

        def fp_body(carry):
            kv, _ = carry
            sup = jnp.dot(kv, s_s[:, :], preferred_element_type=jnp.float32)
            knew = alive0 * jnp.where(sup == 0.0, 1.0, 0.0)
            return knew, jnp.any(knew != kv)

        kfin, _ = jax.lax.while_loop(fp_cond, fp_body, (alive0, True))

        rank = jnp.dot(kfin, p_s[:, :], preferred_element_type=jnp.float32)
        kc = kcnt_ref[0, r, :].astype(jnp.float32)
        f = kfin * jnp.where(rank + kc < MAXD, 1.0, 0.0)
        n_new = jnp.sum(f)
        tslot = rank + kc
        oneh = jnp.where((tslot == iota300) & (f != 0.0), 1.0, 0.0)
        scT_f = jnp.where(scT == neg_inf, 0.0, scT)
        payload = jnp.concatenate([idxT, scT_f, x1T, y1T, x2T, y2T], axis=1)
        newv18 = jnp.dot(oneh, _split_cols(payload),
                         preferred_element_type=jnp.float32)
        newv = newv18[:, 0:6] + newv18[:, 6:12] + newv18[:, 12:18]
        wmask = jnp.sum(oneh, axis=1, keepdims=True) > 0.0

        def upd(out_ref, in_ref, col):
            old = in_ref[0, r, :]
            new = jnp.reshape(newv[:, col:col + 1], (1, MAXD))
            wrow = jnp.reshape(wmask, (1, MAXD))
            out_ref[0, r, :] = jnp.where(wrow, new, old)

        upd(oidx_ref, kidx_ref, 0)
        upd(osc_ref, ksc_ref, 1)
        upd(ox1_ref, kx1_ref, 2)
        upd(oy1_ref, ky1_ref, 3)
        upd(ox2_ref, kx2_ref, 4)
        upd(oy2_ref, ky2_ref, 5)
        kcn = kcnt_ref[0, r, :] + n_new.astype(jnp.int32)
        ocnt_ref[0, r, :] = kcn
        rem = rem_ref[0, r, :]
        done_ref[0, r, :] = ((kcn >= MAXD) | (rem == 0)).astype(jnp.int32)
        return 0

    jax.lax.fori_loop(0, C, cls_body, 0)


def _m_body(kidx_ref, ksc_ref, ksb_ref, kx1_ref, ky1_ref, kx2_ref, ky2_ref,
            ob_ref, os_ref, ol_ref,
            pos_s, sel_s, tp_s, acc_s):
    i32 = jnp.int32
    f32 = jnp.float32
    sbits = ksb_ref[0, :, :]
    kidxf = kidx_ref[0, :, :]
    ciota = jax.lax.broadcasted_iota(i32, (C, MAXD), 0).astype(f32)
    tp_s[:, :] = ciota * jnp.float32(N) + kidxf
    valid = sbits > TH0

    nv = jnp.sum(valid.astype(i32)).reshape(1, 1)

    def bs_body(_, lohi):
        lo, hicur = lohi
        mid = (lo + hicur) // 2
        fmid = jnp.sum((sbits > mid).astype(i32)).reshape(1, 1)
        take = fmid < MAXD
        return jnp.where(take, lo, mid + 1), jnp.where(take, mid, hicur)

    lo0 = jnp.full((1, 1), TH0, i32)
    hi0 = jnp.full((1, 1), HI0, i32)
    _, tbits = jax.lax.fori_loop(0, 30, bs_body, (lo0, hi0))
    small = nv <= MAXD
    tbits = jnp.where(small, TH0, tbits)
    mask_hi = sbits > tbits
    n_hi = jnp.sum(mask_hi.astype(i32)).reshape(1, 1)
    n_hi = jnp.where(small, nv, n_hi)
    n_eq = jnp.where(small, 0, MAXD - n_hi)
    mask_eq = valid & (sbits == tbits)

    def cmprefix(m):
        le = _cumsum_lanes(m)
        row_tot = le[:, MAXD - 1:MAXD]
        ro = row_tot
        kk = 1
        while kk < C:
            sh = jnp.concatenate(
                [jnp.zeros((kk, 1), i32), ro[: C - kk, :]], axis=0)
            ro = ro + sh
            kk *= 2
        return le - m + (ro - row_tot)

    eqrank = cmprefix(mask_eq.astype(i32))
    sel = mask_hi | (mask_eq & (eqrank < n_eq))
    seli = sel.astype(i32)
    pos_s[:, :] = cmprefix(seli)
    sel_s[:, :] = seli
    acc_s[:, :] = jnp.zeros((MK, 21), f32)

    iota_mk = jax.lax.broadcasted_iota(i32, (MK, 1), 0)
    ones_col = jnp.ones((MAXD, 1), f32)

    def cls_body(c, _):
        r = pl.ds(c, 1)
        posc = pos_s[r, :]
        selc = sel_s[r, :] != 0
        oneh = ((posc == iota_mk) & selc).astype(f32)
        scrow = ksc_ref[0, r, :]
        scrow = jnp.where(scrow == jnp.float32(-jnp.inf), 0.0, scrow)
        payload = jnp.concatenate(
            [jnp.reshape(scrow, (MAXD, 1)),
             jnp.reshape(kx1_ref[0, r, :], (MAXD, 1)),
             jnp.reshape(ky1_ref[0, r, :], (MAXD, 1)),
             jnp.reshape(kx2_ref[0, r, :], (MAXD, 1)),
             jnp.reshape(ky2_ref[0, r, :], (MAXD, 1)),
             jnp.reshape(tp_s[r, :], (MAXD, 1)),
             ones_col], axis=1)
        acc_s[:, :] = acc_s[:, :] + jnp.dot(
            oneh, _split_cols(payload), preferred_element_type=f32)
        return 0

    jax.lax.fori_loop(0, C, cls_body, 0)

    acc21 = acc_s[:, :]
    acc = acc21[:, 0:7] + acc21[:, 7:14] + acc21[:, 14:21]
    csT = acc[:, 0:1]
    ctpT = acc[:, 5:6]
    cvaT = acc[:, 6:7] > 0.5
    cs = jnp.reshape(csT, (1, MK))
    ctp = jnp.reshape(ctpT, (1, MK))
    cva = jnp.reshape(cvaT, (1, MK))
    prec = (csT > cs) | ((csT == cs) & (ctpT < ctp))
    pf = jnp.where(prec & cvaT, 1.0, 0.0)
    rank = jnp.sum(pf, axis=0, keepdims=True)
    iota300 = jax.lax.broadcasted_iota(i32, (MAXD, 1), 0).astype(f32)
    oneh2 = jnp.where((rank == iota300) & cva, 1.0, 0.0)
    outs21 = jnp.dot(oneh2, acc21, preferred_element_type=f32)
    outs = outs21[:, 0:7] + outs21[:, 7:14] + outs21[:, 14:21]
    wm = jnp.sum(oneh2, axis=1, keepdims=True) > 0.0

    os_ref[0, :, :] = jnp.reshape(jnp.where(wm, outs[:, 0:1], -1.0), (1, MAXD))
    lab = outs[:, 5:6].astype(i32) // N
    ol_ref[0, :, :] = jnp.reshape(jnp.where(wm, lab, -1), (1, MAXD))
    ob_ref[0, :, 0:1] = jnp.where(wm, outs[:, 1:2], -1.0)
    ob_ref[0, :, 1:2] = jnp.where(wm, outs[:, 2:3], -1.0)
    ob_ref[0, :, 2:3] = jnp.where(wm, outs[:, 3:4], -1.0)
    ob_ref[0, :, 3:4] = jnp.where(wm, outs[:, 4:5], -1.0)


def kernel(boxes, classification):
    B = boxes.shape[0]
    boxes_t = boxes.transpose(0, 2, 1)
    scores_t = classification.transpose(0, 2, 1)
    bits_t = jax.lax.bitcast_convert_type(scores_t, jnp.int32)

    def bspec(shape):
        return pl.BlockSpec((1,) + shape,
                            lambda b: (b,) + (0,) * len(shape))

    f32 = jnp.float32
    i32 = jnp.int32

    ac = pl.pallas_call(
        _ac_body,
        grid=(B,),
        in_specs=[bspec((C, N)), bspec((C, 1)), bspec((C, 1))],
        out_specs=[bspec((C, 1))] * 4,
        out_shape=[jax.ShapeDtypeStruct((B, C, 1), i32)] * 4,
        interpret=_INTERPRET,
    )

    BC = B * C
    sc_compact = _make_sc_compact(BC)
    scores_flat = scores_t.reshape(BC, N)
    pay = jnp.concatenate(
        [scores_t[..., None],
         jnp.broadcast_to(boxes[:, None, :, :], (B, C, N, 4)),
         jnp.zeros((B, C, N, 11), f32)], axis=-1).reshape(BC, N, 16)

    tk = pl.pallas_call(
        _t_body,
        grid=(B,),
        in_specs=[bspec((C, K))] * 6 + [bspec((C, 1))]
        + [bspec((C, MAXD))] * 6 + [bspec((C, 1))],
        out_specs=[bspec((C, MAXD))] * 6 + [bspec((C, 1))] * 2,
        out_shape=[jax.ShapeDtypeStruct((B, C, MAXD), f32)] * 6
        + [jax.ShapeDtypeStruct((B, C, 1), i32)] * 2,
        scratch_shapes=[pltpu.VMEM((K, K), f32), pltpu.VMEM((K, K), f32)],
        interpret=_INTERPRET,
    )

    mg = pl.pallas_call(
        _m_body,
        grid=(B,),
        in_specs=[bspec((C, MAXD))] * 7,
        out_specs=[bspec((MAXD, 4)), bspec((1, MAXD)), bspec((1, MAXD))],
        out_shape=[jax.ShapeDtypeStruct((B, MAXD, 4), f32),
                   jax.ShapeDtypeStruct((B, 1, MAXD), f32),
                   jax.ShapeDtypeStruct((B, 1, MAXD), i32)],
        scratch_shapes=[pltpu.VMEM((C, MAXD), i32), pltpu.VMEM((C, MAXD), i32),
                        pltpu.VMEM((C, MAXD), f32), pltpu.VMEM((MK, 21), f32)],
        interpret=_INTERPRET,
    )

    neg_inf = jnp.float32(-jnp.inf)
    hi = jnp.full((B, C, 1), HI0, i32)
    wm = jnp.zeros((B, C, 1), i32)
    done = jnp.zeros((B, C, 1), i32)
    kplane = jnp.zeros((B, C, MAXD), f32)
    kept0 = (kplane, jnp.full((B, C, MAXD), neg_inf, f32),
             kplane, kplane, kplane, kplane)
    kcnt = jnp.zeros((B, C, 1), i32)

    def cond(st):
        return jnp.any(st[2] == 0)

    def body(st):
        hi, wm, done, kept, kcnt = st
        thi, twm, rem, neq = ac(bits_t, hi, wm)
        par = jnp.repeat(
            jnp.stack([thi.reshape(BC), hi.reshape(BC),
                       wm.reshape(BC), neq.reshape(BC)], axis=1), 16, axis=1)
        par = jnp.pad(par, ((0, 48 - BC), (0, 0)))
        cidxf, payg, ncnt = sc_compact(scores_flat, pay, par)
        cidx = cidxf.reshape(B, C, K)
        valid = jnp.arange(K)[None, :] < ncnt[:, 0:1]
        csc = jnp.where(valid, payg[..., 0],
                        jnp.float32(-jnp.inf)).reshape(B, C, K)
        cb = payg.reshape(B, C, K, 16)
        cx1 = cb[..., 1]; cy1 = cb[..., 2]; cx2 = cb[..., 3]; cy2 = cb[..., 4]
        outs = tk(cidx, csc, cx1, cy1, cx2, cy2, rem,
                  kept[0], kept[1], kept[2], kept[3], kept[4], kept[5], kcnt)
        nkept = tuple(outs[0:6])
        nkcnt, ndone = outs[6], outs[7]
        return (thi, twm, ndone, nkept, nkcnt)

    hi, wm, done, kept, kcnt = jax.lax.while_loop(
        cond, body, (hi, wm, done, kept0, kcnt))

    ksb = jax.lax.bitcast_convert_type(kept[1], i32)
    ob, osc, ol = mg(kept[0], kept[1], ksb, kept[2], kept[3], kept[4], kept[5])
    return ob, osc.reshape(B, MAXD), ol.reshape(B, MAXD)


# speedup vs baseline: 5.3583x; 1.0002x over previous
import functools

import jax
import jax.numpy as jnp
from jax import lax
from jax.experimental import pallas as pl
from jax.experimental.pallas import tpu as pltpu
from jax.experimental.pallas import tpu_sc as plsc

N = 20000
C = 20
MAXD = 300
K = 512
MK = 384
BLK = 2000
SCORE_THR = 0.05
NMS_THR = 0.5
TH0 = 0x3D4CCCCD
HI0 = 0x3F800000

def _cumsum_lanes(x):
    n = x.shape[-1]
    k = 1
    while k < n:
        shifted = jnp.concatenate(
            [jnp.zeros(x.shape[:-1] + (k,), x.dtype), x[..., : n - k]], axis=-1)
        x = x + shifted
        k *= 2
    return x


def _bf16_split3(v):
    h = v.astype(jnp.bfloat16).astype(jnp.float32)
    r = v - h
    m = r.astype(jnp.bfloat16).astype(jnp.float32)
    return h, m, r - m


def _split_cols(p):
    h, m, l = _bf16_split3(p)
    return jnp.concatenate([h, m, l], axis=1)


def _ac_body(bits_ref, hi_ref, wm_ref,
             thi_ref, twm_ref, rem_ref, neq_ref):
    bits = bits_ref[0, :, :]
    hi = hi_ref[0, :, :]
    wm = wm_ref[0, :, :]
    iota_n = jax.lax.broadcasted_iota(jnp.int32, (C, N), 1)
    cand = bits > TH0
    restricted = cand & ((bits < hi) | ((bits == hi) & (iota_n >= wm)))
    ri = restricted.astype(jnp.int32)
    cntr = jnp.sum(ri, axis=1, keepdims=True)

    def bs_body(_, lohi):
        lo, hicur = lohi
        mid = (lo + hicur) // 2
        fmid = jnp.sum((restricted & (bits > mid)).astype(jnp.int32),
                       axis=1, keepdims=True)
        take = fmid < K
        return jnp.where(take, lo, mid + 1), jnp.where(take, mid, hicur)

    lo0 = jnp.full((C, 1), TH0, jnp.int32)
    hi0 = jnp.full((C, 1), HI0, jnp.int32)
    _, tbits = jax.lax.fori_loop(0, 30, bs_body, (lo0, hi0))

    small = cntr <= K
    tbits = jnp.where(small, TH0, tbits)
    mask_hi = restricted & (bits > tbits)
    n_hi = jnp.sum(mask_hi.astype(jnp.int32), axis=1, keepdims=True)
    n_hi = jnp.where(small, cntr, n_hi)
    n_eq_take = jnp.where(small, 0, K - n_hi)

    mask_eq = restricted & (bits == tbits)
    eq_rank = _cumsum_lanes(mask_eq.astype(jnp.int32)) - mask_eq.astype(jnp.int32)
    eq_take = mask_eq & (eq_rank < n_eq_take)
    chunk = mask_hi | eq_take
    n_chunk = n_hi + n_eq_take
    next_wm = jnp.max(jnp.where(eq_take, iota_n, -1), axis=1, keepdims=True) + 1

    thi_ref[0, :, :] = tbits
    twm_ref[0, :, :] = next_wm
    rem_ref[0, :, :] = cntr - n_chunk

    neq_ref[0, :, :] = n_eq_take


_NC = 2
_NS = 16
_NW = _NC * _NS
_KPAD = K + 32


@functools.lru_cache(maxsize=None)
def _make_sc_compact(BC):
    mesh = plsc.VectorSubcoreMesh(core_axis_name="c", subcore_axis_name="s",
                                  num_cores=_NC)
    f32 = jnp.float32
    i32 = jnp.int32

    @functools.partial(
        pl.kernel, mesh=mesh,
        compiler_params=pltpu.CompilerParams(needs_layout_passes=False,
                                             use_tc_tiling_on_sc=False),
        out_type=[jax.ShapeDtypeStruct((BC, K), f32),
                  jax.ShapeDtypeStruct((BC, K, 16), f32),
                  jax.ShapeDtypeStruct((BC, 16), i32)],
        scratch_types=[pltpu.VMEM((N,), f32),
                       pltpu.VMEM((_KPAD,), i32),
                       pltpu.VMEM((K,), i32),
                       pltpu.VMEM((K,), f32),
                       pltpu.VMEM((K, 16), f32),
                       pltpu.VMEM((64,), i32),
                       pltpu.SemaphoreType.DMA],
    )
    def sc_compact(scores_hbm, pay_hbm, par_hbm,
                   cidx_out, pay_out, ncnt_out,
                   sc_v, ci_v, giv, cif_v, pg_v, pv, sem):
        wid = lax.axis_index("s") * _NC + lax.axis_index("c")
        iota16 = lax.broadcasted_iota(i32, (16,), 0)
        zero16 = jnp.zeros((16,), i32)
        th016 = jnp.full((16,), TH0, i32)

        def process(t):
            pltpu.sync_copy(scores_hbm.at[t], sc_v)
            pltpu.sync_copy(par_hbm.at[t], pv)
            tb = pv[pl.ds(0, 16)]
            hi = pv[pl.ds(16, 16)]
            wm = pv[pl.ds(32, 16)]
            ne = pv[pl.ds(48, 16)]

            def initb(i, _):
                ci_v[pl.ds(i * 16, 16)] = zero16
                return 0
            lax.fori_loop(0, _KPAD // 16, initb, 0)

            def p1(i, cnt):
                v = plsc.bitcast(sc_v[pl.ds(i * 16, 16)], i32)
                gidx = iota16 + i * 16
                m = (v > tb) & ((v < hi) | ((v == hi) & (gidx >= wm)))
                plsc.store_compressed(ci_v.at[pl.ds(cnt, 16)], gidx, mask=m)
                return cnt + jnp.sum(m.astype(i32))
            cnt1 = lax.fori_loop(0, N // 16, p1, jnp.int32(0))

            def p2(i, cnt):
                v = plsc.bitcast(sc_v[pl.ds(i * 16, 16)], i32)
                gidx = iota16 + i * 16
                me = ((v > th016) & (v == tb)
                      & ((v < hi) | ((v == hi) & (gidx >= wm))))
                pre = plsc.cumsum(me.astype(i32))
                take = me & ((cnt - cnt1 + pre) <= ne)
                plsc.store_compressed(ci_v.at[pl.ds(cnt, 16)], gidx, mask=take)
                return cnt + jnp.sum(take.astype(i32))
            cnt2 = lax.fori_loop(0, N // 16, p2, cnt1)

            def g1(j, _):
                iv = ci_v[pl.ds(j * 16, 16)]
                cif_v[pl.ds(j * 16, 16)] = iv.astype(f32)
                giv[pl.ds(j * 16, 16)] = iv
                return 0
            lax.fori_loop(0, K // 16, g1, 0)

            pltpu.async_copy(pay_hbm.at[t].at[giv], pg_v, sem).wait()
            pltpu.sync_copy(cif_v, cidx_out.at[t])
            pltpu.sync_copy(pg_v, pay_out.at[t])
            ncv = jnp.full((16,), cnt2, i32)
            ci_v[pl.ds(0, 16)] = ncv
            pltpu.sync_copy(ci_v.at[pl.ds(0, 16)], ncnt_out.at[t])

        process(wid)

        @pl.when(wid + _NW < BC)
        def _second():
            process(wid + _NW)

    return sc_compact


def _t_body(cidx_ref, csc_ref, cx1_ref, cy1_ref, cx2_ref, cy2_ref, rem_ref,
            kidx_ref, ksc_ref, kx1_ref, ky1_ref, kx2_ref, ky2_ref, kcnt_ref,
            oidx_ref, osc_ref, ox1_ref, oy1_ref, ox2_ref, oy2_ref,
            ocnt_ref, done_ref,
            s_s, p_s):
    neg_inf = jnp.float32(-jnp.inf)
    iota300 = jax.lax.broadcasted_iota(jnp.int32, (MAXD, 1), 0).astype(jnp.float32)

    def cls_body(c, _):
        r = pl.ds(c, 1)
        sc = csc_ref[0, r, :]
        idxf = cidx_ref[0, r, :]
        x1 = cx1_ref[0, r, :]; y1 = cy1_ref[0, r, :]
        x2 = cx2_ref[0, r, :]; y2 = cy2_ref[0, r, :]
        area = jnp.maximum(x2 - x1, 0.0) * jnp.maximum(y2 - y1, 0.0)
        scT = jnp.reshape(sc, (K, 1))
        idxT = jnp.reshape(idxf, (K, 1))
        x1T = jnp.reshape(x1, (K, 1)); y1T = jnp.reshape(y1, (K, 1))
        x2T = jnp.reshape(x2, (K, 1)); y2T = jnp.reshape(y2, (K, 1))
        areaT = jnp.reshape(area, (K, 1))

        kx1 = kx1_ref[0, r, :]; ky1 = ky1_ref[0, r, :]
        kx2 = kx2_ref[0, r, :]; ky2 = ky2_ref[0, r, :]
        karea = jnp.maximum(kx2 - kx1, 0.0) * jnp.maximum(ky2 - ky1, 0.0)
        xx1 = jnp.maximum(kx1, x1T); yy1 = jnp.maximum(ky1, y1T)
        xx2 = jnp.minimum(kx2, x2T); yy2 = jnp.minimum(ky2, y2T)
        w = jnp.maximum(xx2 - xx1, 0.0); h = jnp.maximum(yy2 - yy1, 0.0)
        inter = w * h
        iouk = inter / (karea + areaT - inter + 1e-9)
        supk = jnp.sum((iouk > NMS_THR).astype(jnp.float32), axis=1,
                       keepdims=True)
        alive0 = jnp.reshape(
            jnp.where((scT != neg_inf) & (supk == 0.0), 1.0, 0.0), (1, K))

        prec = (scT > sc) | ((scT == sc) & (idxT < idxf))
        xx1c = jnp.maximum(x1T, x1); yy1c = jnp.maximum(y1T, y1)
        xx2c = jnp.minimum(x2T, x2); yy2c = jnp.minimum(y2T, y2)
        wc = jnp.maximum(xx2c - xx1c, 0.0); hc = jnp.maximum(yy2c - yy1c, 0.0)
        interc = wc * hc
        iouc = interc / (areaT + area - interc + 1e-9)
        s_s[:, :] = jnp.where(prec & (iouc > NMS_THR), 1.0, 0.0)
        p_s[:, :] = jnp.where(prec, 1.0, 0.0)

        def fp_cond(carry):
            _, changed = carry
            return changed

        def fp_body(carry):
            kv, _ = carry
            sup = jnp.dot(kv, s_s[:, :], preferred_element_type=jnp.float32)
            knew = alive0 * jnp.where(sup == 0.0, 1.0, 0.0)
            return knew, jnp.any(knew != kv)

        kfin, _ = jax.lax.while_loop(fp_cond, fp_body, (alive0, True))

        rank = jnp.dot(kfin, p_s[:, :], preferred_element_type=jnp.float32)
        kc = kcnt_ref[0, r, :].astype(jnp.float32)
        f = kfin * jnp.where(rank + kc < MAXD, 1.0, 0.0)
        n_new = jnp.sum(f)
        tslot = rank + kc
        oneh = jnp.where((tslot == iota300) & (f != 0.0), 1.0, 0.0)
        scT_f = jnp.where(scT == neg_inf, 0.0, scT)
        payload = jnp.concatenate([idxT, scT_f, x1T, y1T, x2T, y2T], axis=1)
        newv18 = jnp.dot(oneh, _split_cols(payload),
                         preferred_element_type=jnp.float32)
        newv = newv18[:, 0:6] + newv18[:, 6:12] + newv18[:, 12:18]
        wmask = jnp.sum(oneh, axis=1, keepdims=True) > 0.0

        def upd(out_ref, in_ref, col):
            old = in_ref[0, r, :]
            new = jnp.reshape(newv[:, col:col + 1], (1, MAXD))
            wrow = jnp.reshape(wmask, (1, MAXD))
            out_ref[0, r, :] = jnp.where(wrow, new, old)

        upd(oidx_ref, kidx_ref, 0)
        upd(osc_ref, ksc_ref, 1)
        upd(ox1_ref, kx1_ref, 2)
        upd(oy1_ref, ky1_ref, 3)
        upd(ox2_ref, kx2_ref, 4)
        upd(oy2_ref, ky2_ref, 5)
        kcn = kcnt_ref[0, r, :] + n_new.astype(jnp.int32)
        ocnt_ref[0, r, :] = kcn
        rem = rem_ref[0, r, :]
        done_ref[0, r, :] = ((kcn >= MAXD) | (rem == 0)).astype(jnp.int32)
        return 0

    jax.lax.fori_loop(0, C, cls_body, 0)


def _m_body(kidx_ref, ksc_ref, ksb_ref, kx1_ref, ky1_ref, kx2_ref, ky2_ref,
            ob_ref, os_ref, ol_ref,
            pos_s, sel_s, tp_s, acc_s):
    i32 = jnp.int32
    f32 = jnp.float32
    sbits = ksb_ref[0, :, :]
    kidxf = kidx_ref[0, :, :]
    ciota = jax.lax.broadcasted_iota(i32, (C, MAXD), 0).astype(f32)
    tp_s[:, :] = ciota * jnp.float32(N) + kidxf
    valid = sbits > TH0

    nv = jnp.sum(valid.astype(i32)).reshape(1, 1)

    def bs_body(_, lohi):
        lo, hicur = lohi
        mid = (lo + hicur) // 2
        fmid = jnp.sum((sbits > mid).astype(i32)).reshape(1, 1)
        take = fmid < MAXD
        return jnp.where(take, lo, mid + 1), jnp.where(take, mid, hicur)

    lo0 = jnp.full((1, 1), TH0, i32)
    hi0 = jnp.full((1, 1), HI0, i32)
    _, tbits = jax.lax.fori_loop(0, 30, bs_body, (lo0, hi0))
    small = nv <= MAXD
    tbits = jnp.where(small, TH0, tbits)
    mask_hi = sbits > tbits
    n_hi = jnp.sum(mask_hi.astype(i32)).reshape(1, 1)
    n_hi = jnp.where(small, nv, n_hi)
    n_eq = jnp.where(small, 0, MAXD - n_hi)
    mask_eq = valid & (sbits == tbits)

    def cmprefix(m):
        le = _cumsum_lanes(m)
        row_tot = le[:, MAXD - 1:MAXD]
        ro = row_tot
        kk = 1
        while kk < C:
            sh = jnp.concatenate(
                [jnp.zeros((kk, 1), i32), ro[: C - kk, :]], axis=0)
            ro = ro + sh
            kk *= 2
        return le - m + (ro - row_tot)

    eqrank = cmprefix(mask_eq.astype(i32))
    sel = mask_hi | (mask_eq & (eqrank < n_eq))
    seli = sel.astype(i32)
    pos_s[:, :] = cmprefix(seli)
    sel_s[:, :] = seli
    acc_s[:, :] = jnp.zeros((MK, 21), f32)

    iota_mk = jax.lax.broadcasted_iota(i32, (MK, 1), 0)
    ones_col = jnp.ones((MAXD, 1), f32)

    def cls_body(c, _):
        r = pl.ds(c, 1)
        posc = pos_s[r, :]
        selc = sel_s[r, :] != 0
        oneh = ((posc == iota_mk) & selc).astype(f32)
        scrow = ksc_ref[0, r, :]
        scrow = jnp.where(scrow == jnp.float32(-jnp.inf), 0.0, scrow)
        payload = jnp.concatenate(
            [jnp.reshape(scrow, (MAXD, 1)),
             jnp.reshape(kx1_ref[0, r, :], (MAXD, 1)),
             jnp.reshape(ky1_ref[0, r, :], (MAXD, 1)),
             jnp.reshape(kx2_ref[0, r, :], (MAXD, 1)),
             jnp.reshape(ky2_ref[0, r, :], (MAXD, 1)),
             jnp.reshape(tp_s[r, :], (MAXD, 1)),
             ones_col], axis=1)
        acc_s[:, :] = acc_s[:, :] + jnp.dot(
            oneh, _split_cols(payload), preferred_element_type=f32)
        return 0

    jax.lax.fori_loop(0, C, cls_body, 0)

    acc21 = acc_s[:, :]
    acc = acc21[:, 0:7] + acc21[:, 7:14] + acc21[:, 14:21]
    csT = acc[:, 0:1]
    ctpT = acc[:, 5:6]
    cvaT = acc[:, 6:7] > 0.5
    cs = jnp.reshape(csT, (1, MK))
    ctp = jnp.reshape(ctpT, (1, MK))
    cva = jnp.reshape(cvaT, (1, MK))
    prec = (csT > cs) | ((csT == cs) & (ctpT < ctp))
    pf = jnp.where(prec & cvaT, 1.0, 0.0)
    rank = jnp.sum(pf, axis=0, keepdims=True)
    iota300 = jax.lax.broadcasted_iota(i32, (MAXD, 1), 0).astype(f32)
    oneh2 = jnp.where((rank == iota300) & cva, 1.0, 0.0)
    outs21 = jnp.dot(oneh2, acc21, preferred_element_type=f32)
    outs = outs21[:, 0:7] + outs21[:, 7:14] + outs21[:, 14:21]
    wm = jnp.sum(oneh2, axis=1, keepdims=True) > 0.0

    os_ref[0, :, :] = jnp.reshape(jnp.where(wm, outs[:, 0:1], -1.0), (1, MAXD))
    lab = outs[:, 5:6].astype(i32) // N
    ol_ref[0, :, :] = jnp.reshape(jnp.where(wm, lab, -1), (1, MAXD))
    ob_ref[0, :, 0:1] = jnp.where(wm, outs[:, 1:2], -1.0)
    ob_ref[0, :, 1:2] = jnp.where(wm, outs[:, 2:3], -1.0)
    ob_ref[0, :, 2:3] = jnp.where(wm, outs[:, 3:4], -1.0)
    ob_ref[0, :, 3:4] = jnp.where(wm, outs[:, 4:5], -1.0)


def kernel(boxes, classification):
    B = boxes.shape[0]
    scores_t = classification.transpose(0, 2, 1)
    bits_t = jax.lax.bitcast_convert_type(scores_t, jnp.int32)

    def bspec(shape):
        return pl.BlockSpec((1,) + shape,
                            lambda b: (b,) + (0,) * len(shape))

    f32 = jnp.float32
    i32 = jnp.int32

    ac = pl.pallas_call(
        _ac_body,
        grid=(B,),
        in_specs=[bspec((C, N)), bspec((C, 1)), bspec((C, 1))],
        out_specs=[bspec((C, 1))] * 4,
        out_shape=[jax.ShapeDtypeStruct((B, C, 1), i32)] * 4,
    )

    BC = B * C
    sc_compact = _make_sc_compact(BC)
    scores_flat = scores_t.reshape(BC, N)
    pay = jnp.concatenate(
        [scores_t[..., None],
         jnp.broadcast_to(boxes[:, None, :, :], (B, C, N, 4)),
         jnp.zeros((B, C, N, 11), f32)], axis=-1).reshape(BC, N, 16)

    tk = pl.pallas_call(
        _t_body,
        grid=(B,),
        in_specs=[bspec((C, K))] * 6 + [bspec((C, 1))]
        + [bspec((C, MAXD))] * 6 + [bspec((C, 1))],
        out_specs=[bspec((C, MAXD))] * 6 + [bspec((C, 1))] * 2,
        out_shape=[jax.ShapeDtypeStruct((B, C, MAXD), f32)] * 6
        + [jax.ShapeDtypeStruct((B, C, 1), i32)] * 2,
        scratch_shapes=[pltpu.VMEM((K, K), f32), pltpu.VMEM((K, K), f32)],
    )

    mg = pl.pallas_call(
        _m_body,
        grid=(B,),
        in_specs=[bspec((C, MAXD))] * 7,
        out_specs=[bspec((MAXD, 4)), bspec((1, MAXD)), bspec((1, MAXD))],
        out_shape=[jax.ShapeDtypeStruct((B, MAXD, 4), f32),
                   jax.ShapeDtypeStruct((B, 1, MAXD), f32),
                   jax.ShapeDtypeStruct((B, 1, MAXD), i32)],
        scratch_shapes=[pltpu.VMEM((C, MAXD), i32), pltpu.VMEM((C, MAXD), i32),
                        pltpu.VMEM((C, MAXD), f32), pltpu.VMEM((MK, 21), f32)],
    )

    neg_inf = jnp.float32(-jnp.inf)
    hi = jnp.full((B, C, 1), HI0, i32)
    wm = jnp.zeros((B, C, 1), i32)
    done = jnp.zeros((B, C, 1), i32)
    kplane = jnp.zeros((B, C, MAXD), f32)
    kept0 = (kplane, jnp.full((B, C, MAXD), neg_inf, f32),
             kplane, kplane, kplane, kplane)
    kcnt = jnp.zeros((B, C, 1), i32)

    def cond(st):
        return jnp.any(st[2] == 0)

    def body(st):
        hi, wm, done, kept, kcnt = st
        thi, twm, rem, neq = ac(bits_t, hi, wm)
        par = jnp.repeat(
            jnp.stack([thi.reshape(BC), hi.reshape(BC),
                       wm.reshape(BC), neq.reshape(BC)], axis=1), 16, axis=1)
        par = jnp.pad(par, ((0, 48 - BC), (0, 0)))
        cidxf, payg, ncnt = sc_compact(scores_flat, pay, par)
        cidx = cidxf.reshape(B, C, K)
        valid = jnp.arange(K)[None, :] < ncnt[:, 0:1]
        csc = jnp.where(valid, payg[..., 0],
                        jnp.float32(-jnp.inf)).reshape(B, C, K)
        cb = payg.reshape(B, C, K, 16)
        cx1 = cb[..., 1]; cy1 = cb[..., 2]; cx2 = cb[..., 3]; cy2 = cb[..., 4]
        outs = tk(cidx, csc, cx1, cy1, cx2, cy2, rem,
                  kept[0], kept[1], kept[2], kept[3], kept[4], kept[5], kcnt)
        nkept = tuple(outs[0:6])
        nkcnt, ndone = outs[6], outs[7]
        return (thi, twm, ndone, nkept, nkcnt)

    hi, wm, done, kept, kcnt = jax.lax.while_loop(
        cond, body, (hi, wm, done, kept0, kcnt))

    ksb = jax.lax.bitcast_convert_type(kept[1], i32)
    ob, osc, ol = mg(kept[0], kept[1], ksb, kept[2], kept[3], kept[4], kept[5])
    return ob, osc.reshape(B, MAXD), ol.reshape(B, MAXD)
